# Initial kernel scaffold; baseline (speedup 1.0000x reference)
import jax
import jax.numpy as jnp
from jax.experimental import pallas as pl


def kernel(x, edge_index, y, W0, b0, W1, b1, W2, b2, W3, b3, p0, p1, p2, U0, c0, U1, c1, U2, c2):
    raise NotImplementedError("write your pallas kernel here")


# speedup vs baseline: 52.6314x; 52.6314x over previous
import functools

import jax
import jax.numpy as jnp
from jax import lax
from jax.experimental import pallas as pl
from jax.experimental.pallas import tpu as pltpu
from jax.experimental.pallas import tpu_sc as plsc

F32 = jnp.float32
I32 = jnp.int32
NC, NS, L = 2, 16, 16
NW = NC * NS
EB = 128
E = 320000
CAP = E // NW
CAPP = 10240

_MESH = dict(core_axis_name="c", subcore_axis_name="s", num_cores=NC,
             num_subcores=NS)


def _mesh():
    return plsc.VectorSubcoreMesh(**_MESH)


def _params():
    return pltpu.CompilerParams(needs_layout_passes=False)


def _fill(ref, rows, val, dtype):
    v = jnp.full((L,), val, dtype)

    def body(i, c):
        ref[pl.ds(i * L, L)] = v
        return c

    lax.fori_loop(0, rows // L, body, 0)


def _fill2(ref, rows, D):
    z = jnp.zeros((L,), F32)

    def body(i, c):
        for j in range(D // L):
            ref[i, pl.ds(j * L, L)] = z
        return c

    lax.fori_loop(0, rows, body, 0)


def _deg_call(mdstp, counts, n_pad):
    rpt = n_pad // NS

    @functools.partial(
        pl.kernel,
        out_type=jax.ShapeDtypeStruct((NC * n_pad,), F32),
        mesh=_mesh(),
        compiler_params=_params(),
        scratch_types=[pltpu.VMEM((EB,), I32), pltpu.VMEM((EB,), F32),
                       pltpu.VMEM((NW * L,), I32), pltpu.VMEM((rpt,), F32),
                       pltpu.VMEM_SHARED((n_pad,), F32)],
    )
    def k(dst_hbm, cnt_hbm, out_hbm, idx_v, ones_v, cnt_v, zbuf, acc):
        cid = lax.axis_index("c")
        sid = lax.axis_index("s")
        wid = sid * NC + cid
        _fill(zbuf, rpt, 0.0, F32)
        pltpu.sync_copy(zbuf, acc.at[pl.ds(sid * rpt, rpt)])
        pltpu.sync_copy(cnt_hbm, cnt_v)
        for j in range(EB // L):
            ones_v[pl.ds(j * L, L)] = jnp.full((L,), 1.0, F32)
        plsc.subcore_barrier()
        nb_w = cnt_v[pl.ds(wid * L, L)][0]

        def body(i, c):
            off = wid * CAPP + i * EB
            pltpu.sync_copy(dst_hbm.at[pl.ds(off, EB)], idx_v)
            pltpu.sync_copy(ones_v, acc.at[idx_v], add=True)
            return c

        lax.fori_loop(0, nb_w, body, 0)
        plsc.subcore_barrier()
        pltpu.sync_copy(acc.at[pl.ds(sid * rpt, rpt)], zbuf)
        pltpu.sync_copy(zbuf, out_hbm.at[pl.ds(cid * n_pad + sid * rpt, rpt)])

    return k(mdstp, counts).reshape(NC, n_pad)


def _msg_call(g, msrcp, mdstp, counts, n_pad, D):
    rpt = n_pad // NS

    @functools.partial(
        pl.kernel,
        out_type=jax.ShapeDtypeStruct((NC * n_pad, D), F32),
        mesh=_mesh(),
        compiler_params=_params(),
        scratch_types=[pltpu.VMEM((EB,), I32), pltpu.VMEM((EB,), I32),
                       pltpu.VMEM((EB, D), F32), pltpu.SemaphoreType.DMA,
                       pltpu.VMEM((NW * L,), I32),
                       pltpu.VMEM((16, D), F32),
                       pltpu.VMEM_SHARED((n_pad, D), F32)],
    )
    def k(g_hbm, s_hbm, d_hbm, cnt_hbm, out_hbm,
          src_v, dst_v, rows_v, sem, cnt_v, zbuf, acc):
        cid = lax.axis_index("c")
        sid = lax.axis_index("s")
        wid = sid * NC + cid
        _fill2(zbuf, 16, D)

        def zbody(c, carry):
            pltpu.sync_copy(zbuf, acc.at[pl.ds(sid * rpt + c * 16, 16)])
            return carry

        lax.fori_loop(0, rpt // 16, zbody, 0)
        pltpu.sync_copy(cnt_hbm, cnt_v)
        plsc.subcore_barrier()
        nb_w = cnt_v[pl.ds(wid * L, L)][0]

        def body(i, c):
            off = wid * CAPP + i * EB
            pltpu.sync_copy(s_hbm.at[pl.ds(off, EB)], src_v)
            pltpu.sync_copy(d_hbm.at[pl.ds(off, EB)], dst_v)
            pltpu.async_copy(g_hbm.at[src_v], rows_v, sem).wait()
            pltpu.sync_copy(rows_v, acc.at[dst_v], add=True)
            return c

        lax.fori_loop(0, nb_w, body, 0)
        plsc.subcore_barrier()
        pltpu.sync_copy(acc.at[pl.ds(sid * rpt, rpt)],
                        out_hbm.at[pl.ds(cid * n_pad + sid * rpt, rpt)])

    return k(g, msrcp, mdstp, counts).reshape(NC, n_pad, D)


def _pool_edges_call(msrcp, mdstp, counts, inv, n_pad, kk, k_pad):
    spread = k_pad - kk

    @functools.partial(
        pl.kernel,
        out_type=(jax.ShapeDtypeStruct((NW * CAPP,), I32),
                  jax.ShapeDtypeStruct((NW * CAPP,), I32),
                  jax.ShapeDtypeStruct((NW * L,), I32)),
        mesh=_mesh(),
        compiler_params=_params(),
        scratch_types=[pltpu.VMEM((n_pad,), I32),
                       pltpu.VMEM((EB,), I32), pltpu.VMEM((EB,), I32),
                       pltpu.VMEM((CAPP,), I32), pltpu.VMEM((CAPP,), I32),
                       pltpu.VMEM((NW * L,), I32), pltpu.VMEM((L,), I32)],
    )
    def k(s_hbm, d_hbm, cnt_hbm, inv_hbm, ms_hbm, md_hbm, cout_hbm,
          inv_v, src_v, dst_v, ms_v, md_v, cnt_v, cb_v):
        cid = lax.axis_index("c")
        sid = lax.axis_index("s")
        wid = sid * NC + cid
        pltpu.sync_copy(inv_hbm, inv_v)
        pltpu.sync_copy(cnt_hbm, cnt_v)
        nb_in = cnt_v[pl.ds(wid * L, L)][0]
        kk_v = jnp.full((L,), kk, I32)
        iota = lax.iota(I32, L)
        junk = kk_v + lax.rem(iota, jnp.full((L,), spread, I32))

        def body(i, off):
            blk = wid * CAPP + i * EB
            pltpu.sync_copy(s_hbm.at[pl.ds(blk, EB)], src_v)
            pltpu.sync_copy(d_hbm.at[pl.ds(blk, EB)], dst_v)
            for j in range(EB // L):
                s16 = src_v[pl.ds(j * L, L)]
                d16 = dst_v[pl.ds(j * L, L)]
                is16 = plsc.load_gather(inv_v, [s16])
                id16 = plsc.load_gather(inv_v, [d16])
                live = (is16 < kk_v) & (id16 < kk_v)
                plsc.store_compressed(ms_v.at[pl.ds(off, L)], is16, mask=live)
                plsc.store_compressed(md_v.at[pl.ds(off, L)], id16, mask=live)
                off = off + jnp.max(plsc.all_reduce_population_count(live))
            return off

        off = lax.fori_loop(0, nb_in, body, 0)
        for t in range(EB // L):
            ms_v[pl.ds(off + t * L, L)] = junk
            md_v[pl.ds(off + t * L, L)] = junk
        nb_out = (off + EB - 1) // EB
        cb_v[pl.ds(0, L)] = jnp.full((L,), 0, I32) + nb_out
        pltpu.sync_copy(ms_v, ms_hbm.at[pl.ds(wid * CAPP, CAPP)])
        pltpu.sync_copy(md_v, md_hbm.at[pl.ds(wid * CAPP, CAPP)])
        pltpu.sync_copy(cb_v, cout_hbm.at[pl.ds(wid * L, L)])

    return k(msrcp, mdstp, counts, inv)


def _inv_call(perm, n_pad, kk, k_pad):
    @functools.partial(
        pl.kernel,
        out_type=jax.ShapeDtypeStruct((n_pad,), I32),
        mesh=_mesh(),
        compiler_params=_params(),
        scratch_types=[pltpu.VMEM((k_pad,), I32), pltpu.VMEM((n_pad,), I32)],
    )
    def k(perm_hbm, out_hbm, perm_v, inv_v):
        cid = lax.axis_index("c")
        sid = lax.axis_index("s")
        wid = sid * NC + cid

        @pl.when(wid == 0)
        def _():
            pltpu.sync_copy(perm_hbm, perm_v)
            fill = jnp.full((L,), kk, I32)

            def fbody(i, c):
                inv_v[pl.ds(i * L, L)] = fill
                return c

            lax.fori_loop(0, n_pad // L, fbody, 0)
            iota = lax.iota(I32, L)

            def sbody(j, c):
                base = j * L
                p16 = perm_v[pl.ds(base, L)]
                vals = iota + base
                mask = vals < kk
                plsc.store_scatter(inv_v, [p16], vals, mask=mask)
                return c

            lax.fori_loop(0, k_pad // L, sbody, 0)
            pltpu.sync_copy(inv_v, out_hbm)

    return k(perm)


def _gather_call(h, perm, k_pad, D):
    nb = k_pad // EB

    @functools.partial(
        pl.kernel,
        out_type=jax.ShapeDtypeStruct((k_pad, D), F32),
        mesh=_mesh(),
        compiler_params=_params(),
        scratch_types=[pltpu.VMEM((EB,), I32), pltpu.VMEM((EB, D), F32),
                       pltpu.SemaphoreType.DMA],
    )
    def k(h_hbm, perm_hbm, out_hbm, idx_v, rows_v, sem):
        cid = lax.axis_index("c")
        sid = lax.axis_index("s")
        wid = sid * NC + cid
        nb_w = (nb - wid + NW - 1) // NW

        def body(i, c):
            off = (wid + i * NW) * EB
            pltpu.sync_copy(perm_hbm.at[pl.ds(off, EB)], idx_v)
            pltpu.async_copy(h_hbm.at[idx_v], rows_v, sem).wait()
            pltpu.sync_copy(rows_v, out_hbm.at[pl.ds(off, EB)])
            return c

        lax.fori_loop(0, nb_w, body, 0)

    return k(h, perm)


def _scatter_call(hb, perm, k_pad, n_pad, D):
    nb = k_pad // EB
    rpt = n_pad // NS

    @functools.partial(
        pl.kernel,
        out_type=jax.ShapeDtypeStruct((n_pad, D), F32),
        mesh=_mesh(),
        compiler_params=_params(),
        scratch_types=[pltpu.VMEM((EB,), I32), pltpu.VMEM((EB, D), F32),
                       pltpu.VMEM((16, D), F32),
                       pltpu.VMEM_SHARED((n_pad, D), F32)],
    )
    def k(hb_hbm, perm_hbm, out_hbm, idx_v, rows_v, zbuf, acc):
        cid = lax.axis_index("c")
        sid = lax.axis_index("s")

        @pl.when(cid == 0)
        def _():
            _fill2(zbuf, 16, D)

            def zbody(c, carry):
                pltpu.sync_copy(zbuf, acc.at[pl.ds(sid * rpt + c * 16, 16)])
                return carry

            lax.fori_loop(0, rpt // 16, zbody, 0)

        plsc.subcore_barrier()

        @pl.when(cid == 0)
        def _():
            nb_w = (nb - sid + NS - 1) // NS

            def body(i, c):
                off = (sid + i * NS) * EB
                pltpu.sync_copy(perm_hbm.at[pl.ds(off, EB)], idx_v)
                pltpu.sync_copy(hb_hbm.at[pl.ds(off, EB)], rows_v)
                pltpu.sync_copy(rows_v, acc.at[idx_v], add=True)
                return c

            lax.fori_loop(0, nb_w, body, 0)

        plsc.subcore_barrier()

        @pl.when(cid == 0)
        def _():
            pltpu.sync_copy(acc.at[pl.ds(sid * rpt, rpt)],
                            out_hbm.at[pl.ds(sid * rpt, rpt)])

    return k(hb, perm)


_BM = 256


def _mm_call(A, W, A2=None, W2=None, C=None, rs=None, dinv=None):
    m_pad, Ka = A.shape
    N = W.shape[1]
    grid = (m_pad // _BM,)
    ins = [A, W]
    specs = [pl.BlockSpec((_BM, Ka), lambda i: (i, 0)),
             pl.BlockSpec((Ka, N), lambda i: (0, 0))]
    if A2 is not None:
        Kb = A2.shape[1]
        ins += [A2, W2]
        specs += [pl.BlockSpec((_BM, Kb), lambda i: (i, 0)),
                  pl.BlockSpec((Kb, N), lambda i: (0, 0))]
    if C is not None:
        ins.append(C)
        specs.append(pl.BlockSpec((_BM, N), lambda i: (i, 0)))
    if rs is not None:
        ins.append(rs)
        specs.append(pl.BlockSpec((_BM, 1), lambda i: (i, 0)))
    if dinv is not None:
        ins.append(dinv)
        specs.append(pl.BlockSpec((_BM, 1), lambda i: (i, 0)))
    out_shape = [jax.ShapeDtypeStruct((m_pad, N), F32)]
    out_specs = [pl.BlockSpec((_BM, N), lambda i: (i, 0))]
    if dinv is not None:
        out_shape.append(jax.ShapeDtypeStruct((m_pad, N), F32))
        out_specs.append(pl.BlockSpec((_BM, N), lambda i: (i, 0)))

    def body(*refs):
        it = iter(refs)
        a_ref = next(it)
        w_ref = next(it)
        a2_ref = next(it) if A2 is not None else None
        w2_ref = next(it) if A2 is not None else None
        c_ref = next(it) if C is not None else None
        rs_ref = next(it) if rs is not None else None
        dv_ref = next(it) if dinv is not None else None
        hp_ref = next(it)
        g_ref = next(it) if dinv is not None else None
        a = a_ref[...]
        if rs_ref is not None:
            a = a * jnp.tanh(rs_ref[...])
        h = jnp.dot(a, w_ref[...], preferred_element_type=F32)
        if a2_ref is not None:
            h = h + jnp.dot(a2_ref[...], w2_ref[...],
                            preferred_element_type=F32)
        if c_ref is not None:
            h = h + c_ref[...]
        hp_ref[...] = h
        if g_ref is not None:
            g_ref[...] = h * dv_ref[...]

    res = pl.pallas_call(
        body, grid=grid, in_specs=specs, out_specs=out_specs,
        out_shape=out_shape)(*ins)
    return res if dinv is not None else res[0]


def _dinv_call(deg_part):
    _, m_pad = deg_part.shape

    def body(dp_ref, dv_ref):
        deg = dp_ref[0:1, :] + dp_ref[1:2, :] + 2.0
        dv_ref[...] = lax.rsqrt(deg)

    out = pl.pallas_call(
        body,
        out_shape=jax.ShapeDtypeStruct((1, m_pad), F32))(deg_part)
    return out.reshape(m_pad, 1)


def _epi_call(part, hp, dinv, b, n_rows, act, pvec=None):
    m_pad, N = hp.shape
    grid = (m_pad // _BM,)
    ins = [part, hp, dinv, b.reshape(1, N)]
    specs = [pl.BlockSpec((NC, _BM, N), lambda i: (0, i, 0)),
             pl.BlockSpec((_BM, N), lambda i: (i, 0)),
             pl.BlockSpec((_BM, 1), lambda i: (i, 0)),
             pl.BlockSpec((1, N), lambda i: (0, 0))]
    out_shape = [jax.ShapeDtypeStruct((m_pad, N), F32)]
    out_specs = [pl.BlockSpec((_BM, N), lambda i: (i, 0))]
    if pvec is not None:
        ins.append(pvec.reshape(N, 1))
        specs.append(pl.BlockSpec((N, 1), lambda i: (0, 0)))
        out_shape.append(jax.ShapeDtypeStruct((m_pad, 1), F32))
        out_specs.append(pl.BlockSpec((_BM, 1), lambda i: (i, 0)))

    def body(*refs):
        if pvec is not None:
            part_ref, hp_ref, dv_ref, b_ref, p_ref, out_ref, sc_ref = refs
        else:
            part_ref, hp_ref, dv_ref, b_ref, out_ref = refs
        i = pl.program_id(0)
        s = part_ref[0, :, :] + part_ref[1, :, :]
        d = dv_ref[...]
        v = d * s + (2.0 * d * d) * hp_ref[...] + b_ref[...]
        if act == "relu":
            v = jnp.maximum(v, 0.0)
        elif act == "sigmoid":
            v = jax.nn.sigmoid(v)
        rid = lax.broadcasted_iota(I32, (_BM, 1), 0) + i * _BM
        v = jnp.where(rid < n_rows, v, 0.0)
        out_ref[...] = v
        if pvec is not None:
            pv = p_ref[...]
            pn = lax.rsqrt(jnp.sum(pv * pv))
            sc_ref[...] = jnp.dot(v, pv, preferred_element_type=F32) * pn

    res = pl.pallas_call(
        body, grid=grid, in_specs=specs, out_specs=out_specs,
        out_shape=out_shape)(*ins)
    return res if pvec is not None else res[0]


NNODE = [10000, 5000, 2500, 1250]
NPAD = [10240, 5120, 2560, 1280]


def kernel(x, edge_index, y, W0, b0, W1, b1, W2, b2, W3, b3, p0, p1, p2,
           U0, c0, U1, c1, U2, c2):
    src0 = edge_index[:, 0]
    dst0 = edge_index[:, 1]
    x_pad = jnp.pad(x, ((0, NPAD[0] - NNODE[0]), (0, 0)))

    padw = CAPP - CAP
    junk0 = NNODE[0] + (jnp.arange(padw, dtype=I32) % (NPAD[0] - NNODE[0]))
    junk0 = jnp.broadcast_to(junk0, (NW, padw))

    def pack0(a):
        return jnp.concatenate([a.reshape(NW, CAP), junk0], axis=1).reshape(-1)

    msrc = [pack0(src0), None, None, None]
    mdst = [pack0(dst0), None, None, None]
    cnts = [jnp.full((NW * L,), CAPP // EB, I32), None, None, None]

    degp0 = _deg_call(mdst[0], cnts[0], NPAD[0])
    dinv0 = _dinv_call(degp0)
    dinvs = [dinv0, None, None, None]
    perms = [None, None, None]
    mems = []

    h = x_pad
    rs = None
    Wd = [(W0, b0), (W1, b1), (W2, b2), (W3, b3)]
    pv = [p0, p1, p2]

    for lvl in range(3):
        Wl, bl = Wd[lvl]
        hp, g = _mm_call(h, Wl, rs=rs, dinv=dinvs[lvl])
        part = _msg_call(g, msrc[lvl], mdst[lvl], cnts[lvl], NPAD[lvl], 128)
        hout, score = _epi_call(part, hp, dinvs[lvl], bl, NNODE[lvl],
                                "relu", pvec=pv[lvl])
        mems.append(hout)
        kk, k_pad = NNODE[lvl + 1], NPAD[lvl + 1]
        vals, perm = lax.top_k(score[:NNODE[lvl], 0], kk)
        perm_p = jnp.pad(perm, (0, k_pad - kk))
        vals_p = jnp.pad(vals, (0, k_pad - kk)).reshape(k_pad, 1)
        perms[lvl] = perm_p
        inv = _inv_call(perm_p, NPAD[lvl], kk, k_pad)
        ms, md, cn = _pool_edges_call(msrc[lvl], mdst[lvl], cnts[lvl], inv,
                                      NPAD[lvl], kk, k_pad)
        msrc[lvl + 1], mdst[lvl + 1], cnts[lvl + 1] = ms, md, cn
        degp = _deg_call(md, cn, k_pad)
        dinvs[lvl + 1] = _dinv_call(degp)
        h = _gather_call(hout, perm_p, k_pad, 128)
        rs = vals_p

    hp, g = _mm_call(h, W3, rs=rs, dinv=dinvs[3])
    part = _msg_call(g, msrc[3], mdst[3], cnts[3], NPAD[3], 128)
    h = _epi_call(part, hp, dinvs[3], b3, NNODE[3], None)

    for lvl, (Uu, cu) in ((2, (U0, c0)), (1, (U1, c1))):
        hb = _mm_call(h, Uu[128:])
        hs = _scatter_call(hb, perms[lvl], NPAD[lvl + 1], NPAD[lvl], 128)
        hp, g = _mm_call(mems[lvl], Uu[:128], C=hs, dinv=dinvs[lvl])
        part = _msg_call(g, msrc[lvl], mdst[lvl], cnts[lvl], NPAD[lvl], 128)
        h = _epi_call(part, hp, dinvs[lvl], cu, NNODE[lvl], "relu")

    U2a = jnp.pad(U2[:128], ((0, 0), (0, 127)))
    U2b = jnp.pad(U2[128:256], ((0, 0), (0, 127)))
    U2c = jnp.pad(U2[256:], ((0, 0), (0, 127)))
    c2p = jnp.pad(c2, (0, 127))
    hb16 = _mm_call(h, U2c)
    hs16 = _scatter_call(hb16, perms[0], NPAD[1], NPAD[0], 128)
    hp16, g16 = _mm_call(mems[0], U2a, A2=x_pad, W2=U2b, C=hs16, dinv=dinv0)
    part16 = _msg_call(g16, msrc[0], mdst[0], cnts[0], NPAD[0], 128)
    out16 = _epi_call(part16, hp16, dinv0, c2p, NNODE[0], "sigmoid")
    return out16[:NNODE[0], 0]


# speedup vs baseline: 65.1690x; 1.2382x over previous
import functools

import jax
import jax.numpy as jnp
from jax import lax
from jax.experimental import pallas as pl
from jax.experimental.pallas import tpu as pltpu
from jax.experimental.pallas import tpu_sc as plsc

F32 = jnp.float32
I32 = jnp.int32
NC, NS, L = 2, 16, 16
NW = NC * NS
EB = 128
E = 320000
CAP = E // NW
CAPP = 12288

_MESH = dict(core_axis_name="c", subcore_axis_name="s", num_cores=NC,
             num_subcores=NS)


def _mesh():
    return plsc.VectorSubcoreMesh(**_MESH)


def _params():
    return pltpu.CompilerParams(needs_layout_passes=False)


def _fill(ref, rows, val, dtype):
    v = jnp.full((L,), val, dtype)

    def body(i, c):
        ref[pl.ds(i * L, L)] = v
        return c

    lax.fori_loop(0, rows // L, body, 0)


def _fill2(ref, rows, D):
    z = jnp.zeros((L,), F32)

    def body(i, c):
        for j in range(D // L):
            ref[i, pl.ds(j * L, L)] = z
        return c

    lax.fori_loop(0, rows, body, 0)


def _deg_call(mdstp, counts, n_pad):
    rpt = n_pad // NS

    @functools.partial(
        pl.kernel,
        out_type=jax.ShapeDtypeStruct((NC * n_pad,), F32),
        mesh=_mesh(),
        compiler_params=_params(),
        scratch_types=[pltpu.VMEM((EB,), I32), pltpu.VMEM((EB,), F32),
                       pltpu.VMEM((NW * L,), I32), pltpu.VMEM((rpt,), F32),
                       pltpu.VMEM_SHARED((n_pad,), F32)],
    )
    def k(dst_hbm, cnt_hbm, out_hbm, idx_v, ones_v, cnt_v, zbuf, acc):
        cid = lax.axis_index("c")
        sid = lax.axis_index("s")
        wid = sid * NC + cid
        _fill(zbuf, rpt, 0.0, F32)
        pltpu.sync_copy(zbuf, acc.at[pl.ds(sid * rpt, rpt)])
        pltpu.sync_copy(cnt_hbm, cnt_v)
        for j in range(EB // L):
            ones_v[pl.ds(j * L, L)] = jnp.full((L,), 1.0, F32)
        plsc.subcore_barrier()
        nb_w = cnt_v[pl.ds(wid * L, L)][0]

        def body(i, c):
            off = wid * CAPP + i * EB
            pltpu.sync_copy(dst_hbm.at[pl.ds(off, EB)], idx_v)
            pltpu.sync_copy(ones_v, acc.at[idx_v], add=True)
            return c

        lax.fori_loop(0, nb_w, body, 0)
        plsc.subcore_barrier()
        pltpu.sync_copy(acc.at[pl.ds(sid * rpt, rpt)], zbuf)
        pltpu.sync_copy(zbuf, out_hbm.at[pl.ds(cid * n_pad + sid * rpt, rpt)])

    return k(mdstp, counts).reshape(NC, n_pad)


def _msg_call(g, msrcp, mdstp, counts, n_pad, D):
    rpt = n_pad // NS
    QB = 24
    RB = CAPP // EB

    @functools.partial(
        pl.kernel,
        out_type=jax.ShapeDtypeStruct((NC * n_pad, D), F32),
        mesh=_mesh(),
        compiler_params=_params(),
        scratch_types=[pltpu.VMEM((QB, EB), I32), pltpu.VMEM((QB, EB), I32),
                       pltpu.VMEM((2 * EB, D), F32), pltpu.SemaphoreType.DMA,
                       pltpu.VMEM((NW * L,), I32),
                       pltpu.VMEM((16, D), F32),
                       pltpu.VMEM_SHARED((n_pad, D), F32)],
    )
    def k(g_hbm, s_hbm, d_hbm, cnt_hbm, out_hbm,
          src_v, dst_v, rows_v, sem, cnt_v, zbuf, acc):
        cid = lax.axis_index("c")
        sid = lax.axis_index("s")
        wid = sid * NC + cid
        _fill2(zbuf, 16, D)

        def zbody(c, carry):
            pltpu.sync_copy(zbuf, acc.at[pl.ds(sid * rpt + c * 16, 16)])
            return carry

        lax.fori_loop(0, rpt // 16, zbody, 0)
        pltpu.sync_copy(cnt_hbm, cnt_v)
        plsc.subcore_barrier()
        nb_w = cnt_v[pl.ds(wid * L, L)][0]
        ngr = (nb_w + QB - 1) // QB

        def group(gi, c):
            base = wid * RB + gi * QB
            pltpu.sync_copy(s_hbm.at[pl.ds(base, QB)], src_v)
            pltpu.sync_copy(d_hbm.at[pl.ds(base, QB)], dst_v)
            nblk = jnp.minimum(QB, nb_w - gi * QB)
            pltpu.async_copy(g_hbm.at[src_v.at[0]],
                             rows_v.at[pl.ds(0, EB)], sem)

            def blk(b, c2):
                slot = lax.rem(b, 2) * EB
                nxt = jnp.minimum(b + 1, nblk - 1)
                pltpu.make_async_copy(g_hbm.at[src_v.at[0]],
                                      rows_v.at[pl.ds(0, EB)], sem).wait()
                pltpu.async_copy(g_hbm.at[src_v.at[nxt]],
                                 rows_v.at[pl.ds(EB - slot, EB)], sem)
                pltpu.sync_copy(rows_v.at[pl.ds(slot, EB)],
                                acc.at[dst_v.at[b]], add=True)
                return c2

            lax.fori_loop(0, nblk, blk, 0)
            pltpu.make_async_copy(g_hbm.at[src_v.at[0]],
                                  rows_v.at[pl.ds(0, EB)], sem).wait()
            return c

        lax.fori_loop(0, ngr, group, 0)
        plsc.subcore_barrier()
        pltpu.sync_copy(acc.at[pl.ds(sid * rpt, rpt)],
                        out_hbm.at[pl.ds(cid * n_pad + sid * rpt, rpt)])

    ms2 = msrcp.reshape(NW * RB, EB)
    md2 = mdstp.reshape(NW * RB, EB)
    return k(g, ms2, md2, counts).reshape(NC, n_pad, D)


def _pool_edges_call(msrcp, mdstp, counts, inv, n_pad, kk, k_pad):
    spread = k_pad - kk

    @functools.partial(
        pl.kernel,
        out_type=(jax.ShapeDtypeStruct((NW * CAPP,), I32),
                  jax.ShapeDtypeStruct((NW * CAPP,), I32),
                  jax.ShapeDtypeStruct((NW * L,), I32)),
        mesh=_mesh(),
        compiler_params=_params(),
        scratch_types=[pltpu.VMEM((n_pad,), I32),
                       pltpu.VMEM((EB,), I32), pltpu.VMEM((EB,), I32),
                       pltpu.VMEM((CAPP,), I32), pltpu.VMEM((CAPP,), I32),
                       pltpu.VMEM((NW * L,), I32), pltpu.VMEM((L,), I32)],
    )
    def k(s_hbm, d_hbm, cnt_hbm, inv_hbm, ms_hbm, md_hbm, cout_hbm,
          inv_v, src_v, dst_v, ms_v, md_v, cnt_v, cb_v):
        cid = lax.axis_index("c")
        sid = lax.axis_index("s")
        wid = sid * NC + cid
        pltpu.sync_copy(inv_hbm, inv_v)
        pltpu.sync_copy(cnt_hbm, cnt_v)
        nb_in = cnt_v[pl.ds(wid * L, L)][0]
        kk_v = jnp.full((L,), kk, I32)
        iota = lax.iota(I32, L)
        junk = kk_v + lax.rem(iota, jnp.full((L,), spread, I32))

        def body(i, off):
            blk = wid * CAPP + i * EB
            pltpu.sync_copy(s_hbm.at[pl.ds(blk, EB)], src_v)
            pltpu.sync_copy(d_hbm.at[pl.ds(blk, EB)], dst_v)
            for j in range(EB // L):
                s16 = src_v[pl.ds(j * L, L)]
                d16 = dst_v[pl.ds(j * L, L)]
                is16 = plsc.load_gather(inv_v, [s16])
                id16 = plsc.load_gather(inv_v, [d16])
                live = (is16 < kk_v) & (id16 < kk_v)
                plsc.store_compressed(ms_v.at[pl.ds(off, L)], is16, mask=live)
                plsc.store_compressed(md_v.at[pl.ds(off, L)], id16, mask=live)
                off = off + jnp.max(plsc.all_reduce_population_count(live))
            return off

        off = lax.fori_loop(0, nb_in, body, 0)
        for t in range(EB // L):
            ms_v[pl.ds(off + t * L, L)] = junk
            md_v[pl.ds(off + t * L, L)] = junk
        nb_out = (off + EB - 1) // EB
        cb_v[pl.ds(0, L)] = jnp.full((L,), 0, I32) + nb_out
        pltpu.sync_copy(ms_v, ms_hbm.at[pl.ds(wid * CAPP, CAPP)])
        pltpu.sync_copy(md_v, md_hbm.at[pl.ds(wid * CAPP, CAPP)])
        pltpu.sync_copy(cb_v, cout_hbm.at[pl.ds(wid * L, L)])

    return k(msrcp, mdstp, counts, inv)


def _inv_call(perm, n_pad, kk, k_pad):
    @functools.partial(
        pl.kernel,
        out_type=jax.ShapeDtypeStruct((n_pad,), I32),
        mesh=_mesh(),
        compiler_params=_params(),
        scratch_types=[pltpu.VMEM((k_pad,), I32), pltpu.VMEM((n_pad,), I32)],
    )
    def k(perm_hbm, out_hbm, perm_v, inv_v):
        cid = lax.axis_index("c")
        sid = lax.axis_index("s")
        wid = sid * NC + cid

        @pl.when(wid == 0)
        def _():
            pltpu.sync_copy(perm_hbm, perm_v)
            fill = jnp.full((L,), kk, I32)

            def fbody(i, c):
                inv_v[pl.ds(i * L, L)] = fill
                return c

            lax.fori_loop(0, n_pad // L, fbody, 0)
            iota = lax.iota(I32, L)

            def sbody(j, c):
                base = j * L
                p16 = perm_v[pl.ds(base, L)]
                vals = iota + base
                mask = vals < kk
                plsc.store_scatter(inv_v, [p16], vals, mask=mask)
                return c

            lax.fori_loop(0, k_pad // L, sbody, 0)
            pltpu.sync_copy(inv_v, out_hbm)

    return k(perm)


def _gather_call(h, perm, k_pad, D):
    nb = k_pad // EB

    @functools.partial(
        pl.kernel,
        out_type=jax.ShapeDtypeStruct((k_pad, D), F32),
        mesh=_mesh(),
        compiler_params=_params(),
        scratch_types=[pltpu.VMEM((EB,), I32), pltpu.VMEM((EB, D), F32),
                       pltpu.SemaphoreType.DMA],
    )
    def k(h_hbm, perm_hbm, out_hbm, idx_v, rows_v, sem):
        cid = lax.axis_index("c")
        sid = lax.axis_index("s")
        wid = sid * NC + cid
        nb_w = (nb - wid + NW - 1) // NW

        def body(i, c):
            off = (wid + i * NW) * EB
            pltpu.sync_copy(perm_hbm.at[pl.ds(off, EB)], idx_v)
            pltpu.async_copy(h_hbm.at[idx_v], rows_v, sem).wait()
            pltpu.sync_copy(rows_v, out_hbm.at[pl.ds(off, EB)])
            return c

        lax.fori_loop(0, nb_w, body, 0)

    return k(h, perm)


def _scatter_call(hb, perm, k_pad, n_pad, D):
    nb = k_pad // EB
    rpt = n_pad // NS

    @functools.partial(
        pl.kernel,
        out_type=jax.ShapeDtypeStruct((n_pad, D), F32),
        mesh=_mesh(),
        compiler_params=_params(),
        scratch_types=[pltpu.VMEM((EB,), I32), pltpu.VMEM((EB, D), F32),
                       pltpu.VMEM((16, D), F32),
                       pltpu.VMEM_SHARED((n_pad, D), F32)],
    )
    def k(hb_hbm, perm_hbm, out_hbm, idx_v, rows_v, zbuf, acc):
        cid = lax.axis_index("c")
        sid = lax.axis_index("s")

        @pl.when(cid == 0)
        def _():
            _fill2(zbuf, 16, D)

            def zbody(c, carry):
                pltpu.sync_copy(zbuf, acc.at[pl.ds(sid * rpt + c * 16, 16)])
                return carry

            lax.fori_loop(0, rpt // 16, zbody, 0)

        plsc.subcore_barrier()

        @pl.when(cid == 0)
        def _():
            nb_w = (nb - sid + NS - 1) // NS

            def body(i, c):
                off = (sid + i * NS) * EB
                pltpu.sync_copy(perm_hbm.at[pl.ds(off, EB)], idx_v)
                pltpu.sync_copy(hb_hbm.at[pl.ds(off, EB)], rows_v)
                pltpu.sync_copy(rows_v, acc.at[idx_v], add=True)
                return c

            lax.fori_loop(0, nb_w, body, 0)

        plsc.subcore_barrier()

        @pl.when(cid == 0)
        def _():
            pltpu.sync_copy(acc.at[pl.ds(sid * rpt, rpt)],
                            out_hbm.at[pl.ds(sid * rpt, rpt)])

    return k(hb, perm)


_BM = 256


def _mm_call(A, W, A2=None, W2=None, C=None, rs=None, dinv=None):
    m_pad, Ka = A.shape
    N = W.shape[1]
    grid = (m_pad // _BM,)
    ins = [A, W]
    specs = [pl.BlockSpec((_BM, Ka), lambda i: (i, 0)),
             pl.BlockSpec((Ka, N), lambda i: (0, 0))]
    if A2 is not None:
        Kb = A2.shape[1]
        ins += [A2, W2]
        specs += [pl.BlockSpec((_BM, Kb), lambda i: (i, 0)),
                  pl.BlockSpec((Kb, N), lambda i: (0, 0))]
    if C is not None:
        ins.append(C)
        specs.append(pl.BlockSpec((_BM, N), lambda i: (i, 0)))
    if rs is not None:
        ins.append(rs)
        specs.append(pl.BlockSpec((_BM, 1), lambda i: (i, 0)))
    if dinv is not None:
        ins.append(dinv)
        specs.append(pl.BlockSpec((_BM, 1), lambda i: (i, 0)))
    out_shape = [jax.ShapeDtypeStruct((m_pad, N), F32)]
    out_specs = [pl.BlockSpec((_BM, N), lambda i: (i, 0))]
    if dinv is not None:
        out_shape.append(jax.ShapeDtypeStruct((m_pad, N), F32))
        out_specs.append(pl.BlockSpec((_BM, N), lambda i: (i, 0)))

    def body(*refs):
        it = iter(refs)
        a_ref = next(it)
        w_ref = next(it)
        a2_ref = next(it) if A2 is not None else None
        w2_ref = next(it) if A2 is not None else None
        c_ref = next(it) if C is not None else None
        rs_ref = next(it) if rs is not None else None
        dv_ref = next(it) if dinv is not None else None
        hp_ref = next(it)
        g_ref = next(it) if dinv is not None else None
        a = a_ref[...]
        if rs_ref is not None:
            a = a * jnp.tanh(rs_ref[...])
        h = jnp.dot(a, w_ref[...], preferred_element_type=F32)
        if a2_ref is not None:
            h = h + jnp.dot(a2_ref[...], w2_ref[...],
                            preferred_element_type=F32)
        if c_ref is not None:
            h = h + c_ref[...]
        hp_ref[...] = h
        if g_ref is not None:
            g_ref[...] = h * dv_ref[...]

    res = pl.pallas_call(
        body, grid=grid, in_specs=specs, out_specs=out_specs,
        out_shape=out_shape)(*ins)
    return res if dinv is not None else res[0]


def _dinv_call(deg_part):
    _, m_pad = deg_part.shape

    def body(dp_ref, dv_ref):
        deg = dp_ref[0:1, :] + dp_ref[1:2, :] + 2.0
        dv_ref[...] = lax.rsqrt(deg)

    out = pl.pallas_call(
        body,
        out_shape=jax.ShapeDtypeStruct((1, m_pad), F32))(deg_part)
    return out.reshape(m_pad, 1)


def _epi_call(part, hp, dinv, b, n_rows, act, pvec=None):
    m_pad, N = hp.shape
    grid = (m_pad // _BM,)
    ins = [part, hp, dinv, b.reshape(1, N)]
    specs = [pl.BlockSpec((NC, _BM, N), lambda i: (0, i, 0)),
             pl.BlockSpec((_BM, N), lambda i: (i, 0)),
             pl.BlockSpec((_BM, 1), lambda i: (i, 0)),
             pl.BlockSpec((1, N), lambda i: (0, 0))]
    out_shape = [jax.ShapeDtypeStruct((m_pad, N), F32)]
    out_specs = [pl.BlockSpec((_BM, N), lambda i: (i, 0))]
    if pvec is not None:
        ins.append(pvec.reshape(N, 1))
        specs.append(pl.BlockSpec((N, 1), lambda i: (0, 0)))
        out_shape.append(jax.ShapeDtypeStruct((m_pad, 1), F32))
        out_specs.append(pl.BlockSpec((_BM, 1), lambda i: (i, 0)))

    def body(*refs):
        if pvec is not None:
            part_ref, hp_ref, dv_ref, b_ref, p_ref, out_ref, sc_ref = refs
        else:
            part_ref, hp_ref, dv_ref, b_ref, out_ref = refs
        i = pl.program_id(0)
        s = part_ref[0, :, :] + part_ref[1, :, :]
        d = dv_ref[...]
        v = d * s + (2.0 * d * d) * hp_ref[...] + b_ref[...]
        if act == "relu":
            v = jnp.maximum(v, 0.0)
        elif act == "sigmoid":
            v = jax.nn.sigmoid(v)
        rid = lax.broadcasted_iota(I32, (_BM, 1), 0) + i * _BM
        v = jnp.where(rid < n_rows, v, 0.0)
        out_ref[...] = v
        if pvec is not None:
            pv = p_ref[...]
            pn = lax.rsqrt(jnp.sum(pv * pv))
            sc_ref[...] = jnp.dot(v, pv, preferred_element_type=F32) * pn

    res = pl.pallas_call(
        body, grid=grid, in_specs=specs, out_specs=out_specs,
        out_shape=out_shape)(*ins)
    return res if pvec is not None else res[0]


NNODE = [10000, 5000, 2500, 1250]
NPAD = [10240, 5120, 2560, 1280]


def kernel(x, edge_index, y, W0, b0, W1, b1, W2, b2, W3, b3, p0, p1, p2,
           U0, c0, U1, c1, U2, c2):
    src0 = edge_index[:, 0]
    dst0 = edge_index[:, 1]
    x_pad = jnp.pad(x, ((0, NPAD[0] - NNODE[0]), (0, 0)))

    padw = CAPP - CAP
    junk0 = NNODE[0] + (jnp.arange(padw, dtype=I32) % (NPAD[0] - NNODE[0]))
    junk0 = jnp.broadcast_to(junk0, (NW, padw))

    def pack0(a):
        return jnp.concatenate([a.reshape(NW, CAP), junk0], axis=1).reshape(-1)

    msrc = [pack0(src0), None, None, None]
    mdst = [pack0(dst0), None, None, None]
    cnts = [jnp.full((NW * L,), (CAP + EB - 1) // EB, I32), None, None, None]

    degp0 = _deg_call(mdst[0], cnts[0], NPAD[0])
    dinv0 = _dinv_call(degp0)
    dinvs = [dinv0, None, None, None]
    perms = [None, None, None]
    mems = []

    h = x_pad
    rs = None
    Wd = [(W0, b0), (W1, b1), (W2, b2), (W3, b3)]
    pv = [p0, p1, p2]

    for lvl in range(3):
        Wl, bl = Wd[lvl]
        hp, g = _mm_call(h, Wl, rs=rs, dinv=dinvs[lvl])
        part = _msg_call(g, msrc[lvl], mdst[lvl], cnts[lvl], NPAD[lvl], 128)
        hout, score = _epi_call(part, hp, dinvs[lvl], bl, NNODE[lvl],
                                "relu", pvec=pv[lvl])
        mems.append(hout)
        kk, k_pad = NNODE[lvl + 1], NPAD[lvl + 1]
        vals, perm = lax.top_k(score[:NNODE[lvl], 0], kk)
        perm_p = jnp.pad(perm, (0, k_pad - kk))
        vals_p = jnp.pad(vals, (0, k_pad - kk)).reshape(k_pad, 1)
        perms[lvl] = perm_p
        inv = _inv_call(perm_p, NPAD[lvl], kk, k_pad)
        ms, md, cn = _pool_edges_call(msrc[lvl], mdst[lvl], cnts[lvl], inv,
                                      NPAD[lvl], kk, k_pad)
        msrc[lvl + 1], mdst[lvl + 1], cnts[lvl + 1] = ms, md, cn
        degp = _deg_call(md, cn, k_pad)
        dinvs[lvl + 1] = _dinv_call(degp)
        h = _gather_call(hout, perm_p, k_pad, 128)
        rs = vals_p

    hp, g = _mm_call(h, W3, rs=rs, dinv=dinvs[3])
    part = _msg_call(g, msrc[3], mdst[3], cnts[3], NPAD[3], 128)
    h = _epi_call(part, hp, dinvs[3], b3, NNODE[3], None)

    for lvl, (Uu, cu) in ((2, (U0, c0)), (1, (U1, c1))):
        hb = _mm_call(h, Uu[128:])
        hs = _scatter_call(hb, perms[lvl], NPAD[lvl + 1], NPAD[lvl], 128)
        hp, g = _mm_call(mems[lvl], Uu[:128], C=hs, dinv=dinvs[lvl])
        part = _msg_call(g, msrc[lvl], mdst[lvl], cnts[lvl], NPAD[lvl], 128)
        h = _epi_call(part, hp, dinvs[lvl], cu, NNODE[lvl], "relu")

    U2a = jnp.pad(U2[:128], ((0, 0), (0, 127)))
    U2b = jnp.pad(U2[128:256], ((0, 0), (0, 127)))
    U2c = jnp.pad(U2[256:], ((0, 0), (0, 127)))
    c2p = jnp.pad(c2, (0, 127))
    hb16 = _mm_call(h, U2c)
    hs16 = _scatter_call(hb16, perms[0], NPAD[1], NPAD[0], 128)
    hp16, g16 = _mm_call(mems[0], U2a, A2=x_pad, W2=U2b, C=hs16, dinv=dinv0)
    part16 = _msg_call(g16, msrc[0], mdst[0], cnts[0], NPAD[0], 128)
    out16 = _epi_call(part16, hp16, dinv0, c2p, NNODE[0], "sigmoid")
    return out16[:NNODE[0], 0]


# speedup vs baseline: 67.6842x; 1.0386x over previous
import functools

import jax
import jax.numpy as jnp
from jax import lax
from jax.experimental import pallas as pl
from jax.experimental.pallas import tpu as pltpu
from jax.experimental.pallas import tpu_sc as plsc

F32 = jnp.float32
I32 = jnp.int32
NC, NS, L = 2, 16, 16
NW = NC * NS
EB = 128
E = 320000
CAP = E // NW
CAPP = 12288

_MESH = dict(core_axis_name="c", subcore_axis_name="s", num_cores=NC,
             num_subcores=NS)


def _mesh():
    return plsc.VectorSubcoreMesh(**_MESH)


def _params():
    return pltpu.CompilerParams(needs_layout_passes=False)


def _fill(ref, rows, val, dtype):
    v = jnp.full((L,), val, dtype)

    def body(i, c):
        ref[pl.ds(i * L, L)] = v
        return c

    lax.fori_loop(0, rows // L, body, 0)


def _fill2(ref, rows, D):
    z = jnp.zeros((L,), F32)

    def body(i, c):
        for j in range(D // L):
            ref[i, pl.ds(j * L, L)] = z
        return c

    lax.fori_loop(0, rows, body, 0)


def _deg_call(mdstp, counts, n_pad):
    rpt = n_pad // NS

    @functools.partial(
        pl.kernel,
        out_type=jax.ShapeDtypeStruct((NC * n_pad,), F32),
        mesh=_mesh(),
        compiler_params=_params(),
        scratch_types=[pltpu.VMEM((24, EB), I32), pltpu.VMEM((EB,), F32),
                       pltpu.VMEM((NW * L,), I32), pltpu.VMEM((rpt,), F32),
                       pltpu.VMEM_SHARED((n_pad,), F32)],
    )
    def k(dst_hbm, cnt_hbm, out_hbm, idx_v, ones_v, cnt_v, zbuf, acc):
        cid = lax.axis_index("c")
        sid = lax.axis_index("s")
        wid = sid * NC + cid
        _fill(zbuf, rpt, 0.0, F32)
        pltpu.sync_copy(zbuf, acc.at[pl.ds(sid * rpt, rpt)])
        pltpu.sync_copy(cnt_hbm, cnt_v)
        for j in range(EB // L):
            ones_v[pl.ds(j * L, L)] = jnp.full((L,), 1.0, F32)
        plsc.subcore_barrier()
        nb_w = cnt_v[pl.ds(wid * L, L)][0]
        ngr = (nb_w + 24 - 1) // 24

        def group(gi, c):
            base = wid * (CAPP // EB) + gi * 24
            pltpu.sync_copy(dst_hbm.at[pl.ds(base, 24)], idx_v)
            nblk = jnp.minimum(24, nb_w - gi * 24)

            def blk(b, c2):
                pltpu.sync_copy(ones_v, acc.at[idx_v.at[b]], add=True)
                return c2

            lax.fori_loop(0, nblk, blk, 0)
            return c

        lax.fori_loop(0, ngr, group, 0)
        plsc.subcore_barrier()
        pltpu.sync_copy(acc.at[pl.ds(sid * rpt, rpt)], zbuf)
        pltpu.sync_copy(zbuf, out_hbm.at[pl.ds(cid * n_pad + sid * rpt, rpt)])

    return k(mdstp.reshape(NW * (CAPP // EB), EB), counts).reshape(NC, n_pad)


def _msg_call(g, msrcp, mdstp, counts, n_pad, D):
    rpt = n_pad // NS
    QB = 24
    RB = CAPP // EB

    @functools.partial(
        pl.kernel,
        out_type=jax.ShapeDtypeStruct((NC * n_pad, D), F32),
        mesh=_mesh(),
        compiler_params=_params(),
        scratch_types=[pltpu.VMEM((QB, EB), I32), pltpu.VMEM((QB, EB), I32),
                       pltpu.VMEM((2 * EB, D), F32), pltpu.SemaphoreType.DMA,
                       pltpu.VMEM((NW * L,), I32),
                       pltpu.VMEM((40, D), F32),
                       pltpu.VMEM_SHARED((n_pad, D), F32)],
    )
    def k(g_hbm, s_hbm, d_hbm, cnt_hbm, out_hbm,
          src_v, dst_v, rows_v, sem, cnt_v, zbuf, acc):
        cid = lax.axis_index("c")
        sid = lax.axis_index("s")
        wid = sid * NC + cid
        _fill2(zbuf, 40, D)

        def zbody(c, carry):
            pltpu.sync_copy(zbuf, acc.at[pl.ds(sid * rpt + c * 40, 40)])
            return carry

        lax.fori_loop(0, rpt // 40, zbody, 0)
        pltpu.sync_copy(cnt_hbm, cnt_v)
        plsc.subcore_barrier()
        nb_w = cnt_v[pl.ds(wid * L, L)][0]
        ngr = (nb_w + QB - 1) // QB

        def group(gi, c):
            base = wid * RB + gi * QB
            pltpu.sync_copy(s_hbm.at[pl.ds(base, QB)], src_v)
            pltpu.sync_copy(d_hbm.at[pl.ds(base, QB)], dst_v)
            nblk = jnp.minimum(QB, nb_w - gi * QB)
            pltpu.async_copy(g_hbm.at[src_v.at[0]],
                             rows_v.at[pl.ds(0, EB)], sem)

            def blk(b, c2):
                slot = lax.rem(b, 2) * EB
                nxt = jnp.minimum(b + 1, nblk - 1)
                pltpu.make_async_copy(g_hbm.at[src_v.at[0]],
                                      rows_v.at[pl.ds(0, EB)], sem).wait()
                pltpu.async_copy(g_hbm.at[src_v.at[nxt]],
                                 rows_v.at[pl.ds(EB - slot, EB)], sem)
                pltpu.sync_copy(rows_v.at[pl.ds(slot, EB)],
                                acc.at[dst_v.at[b]], add=True)
                return c2

            lax.fori_loop(0, nblk, blk, 0)
            pltpu.make_async_copy(g_hbm.at[src_v.at[0]],
                                  rows_v.at[pl.ds(0, EB)], sem).wait()
            return c

        lax.fori_loop(0, ngr, group, 0)
        plsc.subcore_barrier()
        pltpu.sync_copy(acc.at[pl.ds(sid * rpt, rpt)],
                        out_hbm.at[pl.ds(cid * n_pad + sid * rpt, rpt)])

    ms2 = msrcp.reshape(NW * RB, EB)
    md2 = mdstp.reshape(NW * RB, EB)
    return k(g, ms2, md2, counts).reshape(NC, n_pad, D)


def _pool_edges_call(msrcp, mdstp, counts, inv, n_pad, kk, k_pad):
    spread = k_pad - kk

    @functools.partial(
        pl.kernel,
        out_type=(jax.ShapeDtypeStruct((NW * CAPP,), I32),
                  jax.ShapeDtypeStruct((NW * CAPP,), I32),
                  jax.ShapeDtypeStruct((NW * L,), I32)),
        mesh=_mesh(),
        compiler_params=_params(),
        scratch_types=[pltpu.VMEM((n_pad,), I32),
                       pltpu.VMEM((EB,), I32), pltpu.VMEM((EB,), I32),
                       pltpu.VMEM((CAPP,), I32), pltpu.VMEM((CAPP,), I32),
                       pltpu.VMEM((NW * L,), I32), pltpu.VMEM((L,), I32)],
    )
    def k(s_hbm, d_hbm, cnt_hbm, inv_hbm, ms_hbm, md_hbm, cout_hbm,
          inv_v, src_v, dst_v, ms_v, md_v, cnt_v, cb_v):
        cid = lax.axis_index("c")
        sid = lax.axis_index("s")
        wid = sid * NC + cid
        pltpu.sync_copy(inv_hbm, inv_v)
        pltpu.sync_copy(cnt_hbm, cnt_v)
        nb_in = cnt_v[pl.ds(wid * L, L)][0]
        kk_v = jnp.full((L,), kk, I32)
        iota = lax.iota(I32, L)
        junk = kk_v + lax.rem(iota, jnp.full((L,), spread, I32))

        def body(i, off):
            blk = wid * CAPP + i * EB
            pltpu.sync_copy(s_hbm.at[pl.ds(blk, EB)], src_v)
            pltpu.sync_copy(d_hbm.at[pl.ds(blk, EB)], dst_v)
            for j in range(EB // L):
                s16 = src_v[pl.ds(j * L, L)]
                d16 = dst_v[pl.ds(j * L, L)]
                is16 = plsc.load_gather(inv_v, [s16])
                id16 = plsc.load_gather(inv_v, [d16])
                live = (is16 < kk_v) & (id16 < kk_v)
                plsc.store_compressed(ms_v.at[pl.ds(off, L)], is16, mask=live)
                plsc.store_compressed(md_v.at[pl.ds(off, L)], id16, mask=live)
                off = off + jnp.max(plsc.all_reduce_population_count(live))
            return off

        off = lax.fori_loop(0, nb_in, body, 0)
        for t in range(EB // L):
            ms_v[pl.ds(off + t * L, L)] = junk
            md_v[pl.ds(off + t * L, L)] = junk
        nb_out = (off + EB - 1) // EB
        cb_v[pl.ds(0, L)] = jnp.full((L,), 0, I32) + nb_out
        pltpu.sync_copy(ms_v, ms_hbm.at[pl.ds(wid * CAPP, CAPP)])
        pltpu.sync_copy(md_v, md_hbm.at[pl.ds(wid * CAPP, CAPP)])
        pltpu.sync_copy(cb_v, cout_hbm.at[pl.ds(wid * L, L)])

    return k(msrcp, mdstp, counts, inv)


def _inv_call(perm, n_pad, kk, k_pad):
    @functools.partial(
        pl.kernel,
        out_type=jax.ShapeDtypeStruct((n_pad,), I32),
        mesh=_mesh(),
        compiler_params=_params(),
        scratch_types=[pltpu.VMEM((k_pad,), I32), pltpu.VMEM((n_pad,), I32)],
    )
    def k(perm_hbm, out_hbm, perm_v, inv_v):
        cid = lax.axis_index("c")
        sid = lax.axis_index("s")
        wid = sid * NC + cid

        @pl.when(wid == 0)
        def _():
            pltpu.sync_copy(perm_hbm, perm_v)
            fill = jnp.full((L,), kk, I32)

            def fbody(i, c):
                inv_v[pl.ds(i * L, L)] = fill
                return c

            lax.fori_loop(0, n_pad // L, fbody, 0)
            iota = lax.iota(I32, L)

            def sbody(j, c):
                base = j * L
                p16 = perm_v[pl.ds(base, L)]
                vals = iota + base
                mask = vals < kk
                plsc.store_scatter(inv_v, [p16], vals, mask=mask)
                return c

            lax.fori_loop(0, k_pad // L, sbody, 0)
            pltpu.sync_copy(inv_v, out_hbm)

    return k(perm)


def _gather_call(h, perm, k_pad, D):
    nb = k_pad // EB

    @functools.partial(
        pl.kernel,
        out_type=jax.ShapeDtypeStruct((k_pad, D), F32),
        mesh=_mesh(),
        compiler_params=_params(),
        scratch_types=[pltpu.VMEM((EB,), I32), pltpu.VMEM((EB, D), F32),
                       pltpu.SemaphoreType.DMA],
    )
    def k(h_hbm, perm_hbm, out_hbm, idx_v, rows_v, sem):
        cid = lax.axis_index("c")
        sid = lax.axis_index("s")
        wid = sid * NC + cid
        nb_w = (nb - wid + NW - 1) // NW

        def body(i, c):
            off = (wid + i * NW) * EB
            pltpu.sync_copy(perm_hbm.at[pl.ds(off, EB)], idx_v)
            pltpu.async_copy(h_hbm.at[idx_v], rows_v, sem).wait()
            pltpu.sync_copy(rows_v, out_hbm.at[pl.ds(off, EB)])
            return c

        lax.fori_loop(0, nb_w, body, 0)

    return k(h, perm)


def _scatter_call(hb, perm, k_pad, n_pad, D):
    nb = k_pad // EB
    rpt = n_pad // NS

    @functools.partial(
        pl.kernel,
        out_type=jax.ShapeDtypeStruct((n_pad, D), F32),
        mesh=_mesh(),
        compiler_params=_params(),
        scratch_types=[pltpu.VMEM((EB,), I32), pltpu.VMEM((EB, D), F32),
                       pltpu.VMEM((40, D), F32),
                       pltpu.VMEM_SHARED((n_pad, D), F32)],
    )
    def k(hb_hbm, perm_hbm, out_hbm, idx_v, rows_v, zbuf, acc):
        cid = lax.axis_index("c")
        sid = lax.axis_index("s")

        @pl.when(cid == 0)
        def _():
            _fill2(zbuf, 40, D)

            def zbody(c, carry):
                pltpu.sync_copy(zbuf, acc.at[pl.ds(sid * rpt + c * 40, 40)])
                return carry

            lax.fori_loop(0, rpt // 40, zbody, 0)

        plsc.subcore_barrier()

        @pl.when(cid == 0)
        def _():
            nb_w = (nb - sid + NS - 1) // NS

            def body(i, c):
                off = (sid + i * NS) * EB
                pltpu.sync_copy(perm_hbm.at[pl.ds(off, EB)], idx_v)
                pltpu.sync_copy(hb_hbm.at[pl.ds(off, EB)], rows_v)
                pltpu.sync_copy(rows_v, acc.at[idx_v], add=True)
                return c

            lax.fori_loop(0, nb_w, body, 0)

        plsc.subcore_barrier()

        @pl.when(cid == 0)
        def _():
            pltpu.sync_copy(acc.at[pl.ds(sid * rpt, rpt)],
                            out_hbm.at[pl.ds(sid * rpt, rpt)])

    return k(hb, perm)


_BM = 256


def _mm_call(A, W, A2=None, W2=None, C=None, rs=None, dinv=None):
    m_pad, Ka = A.shape
    N = W.shape[1]
    grid = (m_pad // _BM,)
    ins = [A, W]
    specs = [pl.BlockSpec((_BM, Ka), lambda i: (i, 0)),
             pl.BlockSpec((Ka, N), lambda i: (0, 0))]
    if A2 is not None:
        Kb = A2.shape[1]
        ins += [A2, W2]
        specs += [pl.BlockSpec((_BM, Kb), lambda i: (i, 0)),
                  pl.BlockSpec((Kb, N), lambda i: (0, 0))]
    if C is not None:
        ins.append(C)
        specs.append(pl.BlockSpec((_BM, N), lambda i: (i, 0)))
    if rs is not None:
        ins.append(rs)
        specs.append(pl.BlockSpec((_BM, 1), lambda i: (i, 0)))
    if dinv is not None:
        ins.append(dinv)
        specs.append(pl.BlockSpec((_BM, 1), lambda i: (i, 0)))
    out_shape = [jax.ShapeDtypeStruct((m_pad, N), F32)]
    out_specs = [pl.BlockSpec((_BM, N), lambda i: (i, 0))]
    if dinv is not None:
        out_shape.append(jax.ShapeDtypeStruct((m_pad, N), F32))
        out_specs.append(pl.BlockSpec((_BM, N), lambda i: (i, 0)))

    def body(*refs):
        it = iter(refs)
        a_ref = next(it)
        w_ref = next(it)
        a2_ref = next(it) if A2 is not None else None
        w2_ref = next(it) if A2 is not None else None
        c_ref = next(it) if C is not None else None
        rs_ref = next(it) if rs is not None else None
        dv_ref = next(it) if dinv is not None else None
        hp_ref = next(it)
        g_ref = next(it) if dinv is not None else None
        a = a_ref[...]
        if rs_ref is not None:
            a = a * jnp.tanh(rs_ref[...])
        h = jnp.dot(a, w_ref[...], preferred_element_type=F32)
        if a2_ref is not None:
            h = h + jnp.dot(a2_ref[...], w2_ref[...],
                            preferred_element_type=F32)
        if c_ref is not None:
            h = h + c_ref[...]
        hp_ref[...] = h
        if g_ref is not None:
            g_ref[...] = h * dv_ref[...]

    res = pl.pallas_call(
        body, grid=grid, in_specs=specs, out_specs=out_specs,
        out_shape=out_shape)(*ins)
    return res if dinv is not None else res[0]


def _dinv_call(deg_part):
    _, m_pad = deg_part.shape

    def body(dp_ref, dv_ref):
        deg = dp_ref[0:1, :] + dp_ref[1:2, :] + 2.0
        dv_ref[...] = lax.rsqrt(deg)

    out = pl.pallas_call(
        body,
        out_shape=jax.ShapeDtypeStruct((1, m_pad), F32))(deg_part)
    return out.reshape(m_pad, 1)


def _epi_call(part, hp, dinv, b, n_rows, act, pvec=None):
    m_pad, N = hp.shape
    grid = (m_pad // _BM,)
    ins = [part, hp, dinv, b.reshape(1, N)]
    specs = [pl.BlockSpec((NC, _BM, N), lambda i: (0, i, 0)),
             pl.BlockSpec((_BM, N), lambda i: (i, 0)),
             pl.BlockSpec((_BM, 1), lambda i: (i, 0)),
             pl.BlockSpec((1, N), lambda i: (0, 0))]
    out_shape = [jax.ShapeDtypeStruct((m_pad, N), F32)]
    out_specs = [pl.BlockSpec((_BM, N), lambda i: (i, 0))]
    if pvec is not None:
        ins.append(pvec.reshape(N, 1))
        specs.append(pl.BlockSpec((N, 1), lambda i: (0, 0)))
        out_shape.append(jax.ShapeDtypeStruct((m_pad, 1), F32))
        out_specs.append(pl.BlockSpec((_BM, 1), lambda i: (i, 0)))

    def body(*refs):
        if pvec is not None:
            part_ref, hp_ref, dv_ref, b_ref, p_ref, out_ref, sc_ref = refs
        else:
            part_ref, hp_ref, dv_ref, b_ref, out_ref = refs
        i = pl.program_id(0)
        s = part_ref[0, :, :] + part_ref[1, :, :]
        d = dv_ref[...]
        v = d * s + (2.0 * d * d) * hp_ref[...] + b_ref[...]
        if act == "relu":
            v = jnp.maximum(v, 0.0)
        elif act == "sigmoid":
            v = jax.nn.sigmoid(v)
        rid = lax.broadcasted_iota(I32, (_BM, 1), 0) + i * _BM
        v = jnp.where(rid < n_rows, v, 0.0)
        out_ref[...] = v
        if pvec is not None:
            pv = p_ref[...]
            pn = lax.rsqrt(jnp.sum(pv * pv))
            sc_ref[...] = jnp.dot(v, pv, preferred_element_type=F32) * pn

    res = pl.pallas_call(
        body, grid=grid, in_specs=specs, out_specs=out_specs,
        out_shape=out_shape)(*ins)
    return res if pvec is not None else res[0]


NNODE = [10000, 5000, 2500, 1250]
NPAD = [10240, 5120, 2560, 1280]


def kernel(x, edge_index, y, W0, b0, W1, b1, W2, b2, W3, b3, p0, p1, p2,
           U0, c0, U1, c1, U2, c2):
    src0 = edge_index[:, 0]
    dst0 = edge_index[:, 1]
    x_pad = jnp.pad(x, ((0, NPAD[0] - NNODE[0]), (0, 0)))

    padw = CAPP - CAP
    junk0 = NNODE[0] + (jnp.arange(padw, dtype=I32) % (NPAD[0] - NNODE[0]))
    junk0 = jnp.broadcast_to(junk0, (NW, padw))

    def pack0(a):
        return jnp.concatenate([a.reshape(NW, CAP), junk0], axis=1).reshape(-1)

    msrc = [pack0(src0), None, None, None]
    mdst = [pack0(dst0), None, None, None]
    cnts = [jnp.full((NW * L,), (CAP + EB - 1) // EB, I32), None, None, None]

    degp0 = _deg_call(mdst[0], cnts[0], NPAD[0])
    dinv0 = _dinv_call(degp0)
    dinvs = [dinv0, None, None, None]
    perms = [None, None, None]
    mems = []

    h = x_pad
    rs = None
    Wd = [(W0, b0), (W1, b1), (W2, b2), (W3, b3)]
    pv = [p0, p1, p2]

    for lvl in range(3):
        Wl, bl = Wd[lvl]
        hp, g = _mm_call(h, Wl, rs=rs, dinv=dinvs[lvl])
        part = _msg_call(g, msrc[lvl], mdst[lvl], cnts[lvl], NPAD[lvl], 128)
        hout, score = _epi_call(part, hp, dinvs[lvl], bl, NNODE[lvl],
                                "relu", pvec=pv[lvl])
        mems.append(hout)
        kk, k_pad = NNODE[lvl + 1], NPAD[lvl + 1]
        vals, perm = lax.top_k(score[:NNODE[lvl], 0], kk)
        perm_p = jnp.pad(perm, (0, k_pad - kk))
        vals_p = jnp.pad(vals, (0, k_pad - kk)).reshape(k_pad, 1)
        perms[lvl] = perm_p
        inv = _inv_call(perm_p, NPAD[lvl], kk, k_pad)
        ms, md, cn = _pool_edges_call(msrc[lvl], mdst[lvl], cnts[lvl], inv,
                                      NPAD[lvl], kk, k_pad)
        msrc[lvl + 1], mdst[lvl + 1], cnts[lvl + 1] = ms, md, cn
        degp = _deg_call(md, cn, k_pad)
        dinvs[lvl + 1] = _dinv_call(degp)
        h = _gather_call(hout, perm_p, k_pad, 128)
        rs = vals_p

    hp, g = _mm_call(h, W3, rs=rs, dinv=dinvs[3])
    part = _msg_call(g, msrc[3], mdst[3], cnts[3], NPAD[3], 128)
    h = _epi_call(part, hp, dinvs[3], b3, NNODE[3], None)

    for lvl, (Uu, cu) in ((2, (U0, c0)), (1, (U1, c1))):
        hb = _mm_call(h, Uu[128:])
        hs = _scatter_call(hb, perms[lvl], NPAD[lvl + 1], NPAD[lvl], 128)
        hp, g = _mm_call(mems[lvl], Uu[:128], C=hs, dinv=dinvs[lvl])
        part = _msg_call(g, msrc[lvl], mdst[lvl], cnts[lvl], NPAD[lvl], 128)
        h = _epi_call(part, hp, dinvs[lvl], cu, NNODE[lvl], "relu")

    U2a = jnp.pad(U2[:128], ((0, 0), (0, 127)))
    U2b = jnp.pad(U2[128:256], ((0, 0), (0, 127)))
    U2c = jnp.pad(U2[256:], ((0, 0), (0, 127)))
    c2p = jnp.pad(c2, (0, 127))
    hb16 = _mm_call(h, U2c)
    hs16 = _scatter_call(hb16, perms[0], NPAD[1], NPAD[0], 128)
    hp16, g16 = _mm_call(mems[0], U2a, A2=x_pad, W2=U2b, C=hs16, dinv=dinv0)
    part16 = _msg_call(g16, msrc[0], mdst[0], cnts[0], NPAD[0], 128)
    out16 = _epi_call(part16, hp16, dinv0, c2p, NNODE[0], "sigmoid")
    return out16[:NNODE[0], 0]


# speedup vs baseline: 67.8426x; 1.0023x over previous
import functools

import jax
import jax.numpy as jnp
from jax import lax
from jax.experimental import pallas as pl
from jax.experimental.pallas import tpu as pltpu
from jax.experimental.pallas import tpu_sc as plsc

F32 = jnp.float32
I32 = jnp.int32
NC, NS, L = 2, 16, 16
NW = NC * NS
EB = 128
E = 320000
CAP = E // NW
CAPP = 12288

_MESH = dict(core_axis_name="c", subcore_axis_name="s", num_cores=NC,
             num_subcores=NS)


def _mesh():
    return plsc.VectorSubcoreMesh(**_MESH)


def _params():
    return pltpu.CompilerParams(needs_layout_passes=False)


def _fill(ref, rows, val, dtype):
    v = jnp.full((L,), val, dtype)

    def body(i, c):
        ref[pl.ds(i * L, L)] = v
        return c

    lax.fori_loop(0, rows // L, body, 0)


def _fill2(ref, rows, D):
    z = jnp.zeros((L,), F32)

    def body(i, c):
        for j in range(D // L):
            ref[i, pl.ds(j * L, L)] = z
        return c

    lax.fori_loop(0, rows, body, 0)


def _deg_call(mdstp, counts, n_pad):
    rpt = n_pad // NS

    @functools.partial(
        pl.kernel,
        out_type=jax.ShapeDtypeStruct((NC * n_pad,), F32),
        mesh=_mesh(),
        compiler_params=_params(),
        scratch_types=[pltpu.VMEM((24, EB), I32), pltpu.VMEM((EB,), F32),
                       pltpu.VMEM((NW * L,), I32), pltpu.VMEM((rpt,), F32),
                       pltpu.VMEM_SHARED((n_pad,), F32)],
    )
    def k(dst_hbm, cnt_hbm, out_hbm, idx_v, ones_v, cnt_v, zbuf, acc):
        cid = lax.axis_index("c")
        sid = lax.axis_index("s")
        wid = sid * NC + cid
        _fill(zbuf, rpt, 0.0, F32)
        pltpu.sync_copy(zbuf, acc.at[pl.ds(sid * rpt, rpt)])
        pltpu.sync_copy(cnt_hbm, cnt_v)
        for j in range(EB // L):
            ones_v[pl.ds(j * L, L)] = jnp.full((L,), 1.0, F32)
        plsc.subcore_barrier()
        nb_w = cnt_v[pl.ds(wid * L, L)][0]
        ngr = (nb_w + 24 - 1) // 24

        def group(gi, c):
            base = wid * (CAPP // EB) + gi * 24
            pltpu.sync_copy(dst_hbm.at[pl.ds(base, 24)], idx_v)
            nblk = jnp.minimum(24, nb_w - gi * 24)

            def blk(b, c2):
                pltpu.sync_copy(ones_v, acc.at[idx_v.at[b]], add=True)
                return c2

            lax.fori_loop(0, nblk, blk, 0)
            return c

        lax.fori_loop(0, ngr, group, 0)
        plsc.subcore_barrier()
        pltpu.sync_copy(acc.at[pl.ds(sid * rpt, rpt)], zbuf)
        pltpu.sync_copy(zbuf, out_hbm.at[pl.ds(cid * n_pad + sid * rpt, rpt)])

    return k(mdstp.reshape(NW * (CAPP // EB), EB), counts).reshape(NC, n_pad)


def _msg_call(g, msrcp, mdstp, counts, n_pad, D):
    rpt = n_pad // NS
    QB = 32
    RB = CAPP // EB

    @functools.partial(
        pl.kernel,
        out_type=jax.ShapeDtypeStruct((NC * n_pad, D), F32),
        mesh=_mesh(),
        compiler_params=_params(),
        scratch_types=[pltpu.VMEM((QB, EB), I32), pltpu.VMEM((QB, EB), I32),
                       pltpu.VMEM((2 * EB, D), F32), pltpu.SemaphoreType.DMA,
                       pltpu.SemaphoreType.DMA,
                       pltpu.VMEM((NW * L,), I32),
                       pltpu.VMEM((40, D), F32),
                       pltpu.VMEM_SHARED((n_pad, D), F32)],
    )
    def k(g_hbm, s_hbm, d_hbm, cnt_hbm, out_hbm,
          src_v, dst_v, rows_v, sem, sem_s, cnt_v, zbuf, acc):
        cid = lax.axis_index("c")
        sid = lax.axis_index("s")
        wid = sid * NC + cid
        _fill2(zbuf, 40, D)

        def zbody(c, carry):
            pltpu.sync_copy(zbuf, acc.at[pl.ds(sid * rpt + c * 40, 40)])
            return carry

        lax.fori_loop(0, rpt // 40, zbody, 0)
        pltpu.sync_copy(cnt_hbm, cnt_v)
        plsc.subcore_barrier()
        nb_w = cnt_v[pl.ds(wid * L, L)][0]
        ngr = (nb_w + QB - 1) // QB

        def group(gi, c):
            base = wid * RB + gi * QB
            pltpu.sync_copy(s_hbm.at[pl.ds(base, QB)], src_v)
            pltpu.sync_copy(d_hbm.at[pl.ds(base, QB)], dst_v)
            nblk = jnp.minimum(QB, nb_w - gi * QB)
            pltpu.async_copy(g_hbm.at[src_v.at[0]],
                             rows_v.at[pl.ds(0, EB)], sem)

            def blk(b, c2):
                slot = lax.rem(b, 2) * EB
                nxt = jnp.minimum(b + 1, nblk - 1)
                pltpu.make_async_copy(g_hbm.at[src_v.at[0]],
                                      rows_v.at[pl.ds(0, EB)], sem).wait()

                @pl.when(b >= 1)
                def _():
                    pltpu.make_async_copy(
                        rows_v.at[pl.ds(0, EB)], acc.at[dst_v.at[0]],
                        sem_s).wait()

                pltpu.async_copy(g_hbm.at[src_v.at[nxt]],
                                 rows_v.at[pl.ds(EB - slot, EB)], sem)
                pltpu.async_copy(rows_v.at[pl.ds(slot, EB)],
                                 acc.at[dst_v.at[b]], sem_s, add=True)
                return c2

            lax.fori_loop(0, nblk, blk, 0)
            pltpu.make_async_copy(g_hbm.at[src_v.at[0]],
                                  rows_v.at[pl.ds(0, EB)], sem).wait()
            pltpu.make_async_copy(rows_v.at[pl.ds(0, EB)],
                                  acc.at[dst_v.at[0]], sem_s).wait()
            return c

        lax.fori_loop(0, ngr, group, 0)
        plsc.subcore_barrier()
        pltpu.sync_copy(acc.at[pl.ds(sid * rpt, rpt)],
                        out_hbm.at[pl.ds(cid * n_pad + sid * rpt, rpt)])

    ms2 = msrcp.reshape(NW * RB, EB)
    md2 = mdstp.reshape(NW * RB, EB)
    return k(g, ms2, md2, counts).reshape(NC, n_pad, D)


def _pool_edges_call(msrcp, mdstp, counts, inv, n_pad, kk, k_pad):
    spread = k_pad - kk

    @functools.partial(
        pl.kernel,
        out_type=(jax.ShapeDtypeStruct((NW * CAPP,), I32),
                  jax.ShapeDtypeStruct((NW * CAPP,), I32),
                  jax.ShapeDtypeStruct((NW * L,), I32)),
        mesh=_mesh(),
        compiler_params=_params(),
        scratch_types=[pltpu.VMEM((n_pad,), I32),
                       pltpu.VMEM((EB,), I32), pltpu.VMEM((EB,), I32),
                       pltpu.VMEM((CAPP,), I32), pltpu.VMEM((CAPP,), I32),
                       pltpu.VMEM((NW * L,), I32), pltpu.VMEM((L,), I32)],
    )
    def k(s_hbm, d_hbm, cnt_hbm, inv_hbm, ms_hbm, md_hbm, cout_hbm,
          inv_v, src_v, dst_v, ms_v, md_v, cnt_v, cb_v):
        cid = lax.axis_index("c")
        sid = lax.axis_index("s")
        wid = sid * NC + cid
        pltpu.sync_copy(inv_hbm, inv_v)
        pltpu.sync_copy(cnt_hbm, cnt_v)
        nb_in = cnt_v[pl.ds(wid * L, L)][0]
        kk_v = jnp.full((L,), kk, I32)
        iota = lax.iota(I32, L)
        junk = kk_v + lax.rem(iota, jnp.full((L,), spread, I32))

        def body(i, off):
            blk = wid * CAPP + i * EB
            pltpu.sync_copy(s_hbm.at[pl.ds(blk, EB)], src_v)
            pltpu.sync_copy(d_hbm.at[pl.ds(blk, EB)], dst_v)
            for j in range(EB // L):
                s16 = src_v[pl.ds(j * L, L)]
                d16 = dst_v[pl.ds(j * L, L)]
                is16 = plsc.load_gather(inv_v, [s16])
                id16 = plsc.load_gather(inv_v, [d16])
                live = (is16 < kk_v) & (id16 < kk_v)
                plsc.store_compressed(ms_v.at[pl.ds(off, L)], is16, mask=live)
                plsc.store_compressed(md_v.at[pl.ds(off, L)], id16, mask=live)
                off = off + jnp.max(plsc.all_reduce_population_count(live))
            return off

        off = lax.fori_loop(0, nb_in, body, 0)
        for t in range(EB // L):
            ms_v[pl.ds(off + t * L, L)] = junk
            md_v[pl.ds(off + t * L, L)] = junk
        nb_out = (off + EB - 1) // EB
        cb_v[pl.ds(0, L)] = jnp.full((L,), 0, I32) + nb_out
        pltpu.sync_copy(ms_v, ms_hbm.at[pl.ds(wid * CAPP, CAPP)])
        pltpu.sync_copy(md_v, md_hbm.at[pl.ds(wid * CAPP, CAPP)])
        pltpu.sync_copy(cb_v, cout_hbm.at[pl.ds(wid * L, L)])

    return k(msrcp, mdstp, counts, inv)


def _inv_call(perm, n_pad, kk, k_pad):
    @functools.partial(
        pl.kernel,
        out_type=jax.ShapeDtypeStruct((n_pad,), I32),
        mesh=_mesh(),
        compiler_params=_params(),
        scratch_types=[pltpu.VMEM((k_pad,), I32), pltpu.VMEM((n_pad,), I32)],
    )
    def k(perm_hbm, out_hbm, perm_v, inv_v):
        cid = lax.axis_index("c")
        sid = lax.axis_index("s")
        wid = sid * NC + cid

        @pl.when(wid == 0)
        def _():
            pltpu.sync_copy(perm_hbm, perm_v)
            fill = jnp.full((L,), kk, I32)

            def fbody(i, c):
                inv_v[pl.ds(i * L, L)] = fill
                return c

            lax.fori_loop(0, n_pad // L, fbody, 0)
            iota = lax.iota(I32, L)

            def sbody(j, c):
                base = j * L
                p16 = perm_v[pl.ds(base, L)]
                vals = iota + base
                mask = vals < kk
                plsc.store_scatter(inv_v, [p16], vals, mask=mask)
                return c

            lax.fori_loop(0, k_pad // L, sbody, 0)
            pltpu.sync_copy(inv_v, out_hbm)

    return k(perm)


def _gather_call(h, perm, k_pad, D):
    nb = k_pad // EB

    @functools.partial(
        pl.kernel,
        out_type=jax.ShapeDtypeStruct((k_pad, D), F32),
        mesh=_mesh(),
        compiler_params=_params(),
        scratch_types=[pltpu.VMEM((EB,), I32), pltpu.VMEM((EB, D), F32),
                       pltpu.SemaphoreType.DMA],
    )
    def k(h_hbm, perm_hbm, out_hbm, idx_v, rows_v, sem):
        cid = lax.axis_index("c")
        sid = lax.axis_index("s")
        wid = sid * NC + cid
        nb_w = (nb - wid + NW - 1) // NW

        def body(i, c):
            off = (wid + i * NW) * EB
            pltpu.sync_copy(perm_hbm.at[pl.ds(off, EB)], idx_v)
            pltpu.async_copy(h_hbm.at[idx_v], rows_v, sem).wait()
            pltpu.sync_copy(rows_v, out_hbm.at[pl.ds(off, EB)])
            return c

        lax.fori_loop(0, nb_w, body, 0)

    return k(h, perm)


def _scatter_call(hb, perm, k_pad, n_pad, D):
    nb = k_pad // EB
    rpt = n_pad // NS

    @functools.partial(
        pl.kernel,
        out_type=jax.ShapeDtypeStruct((n_pad, D), F32),
        mesh=_mesh(),
        compiler_params=_params(),
        scratch_types=[pltpu.VMEM((EB,), I32), pltpu.VMEM((EB, D), F32),
                       pltpu.VMEM((40, D), F32),
                       pltpu.VMEM_SHARED((n_pad, D), F32)],
    )
    def k(hb_hbm, perm_hbm, out_hbm, idx_v, rows_v, zbuf, acc):
        cid = lax.axis_index("c")
        sid = lax.axis_index("s")

        @pl.when(cid == 0)
        def _():
            _fill2(zbuf, 40, D)

            def zbody(c, carry):
                pltpu.sync_copy(zbuf, acc.at[pl.ds(sid * rpt + c * 40, 40)])
                return carry

            lax.fori_loop(0, rpt // 40, zbody, 0)

        plsc.subcore_barrier()

        @pl.when(cid == 0)
        def _():
            nb_w = (nb - sid + NS - 1) // NS

            def body(i, c):
                off = (sid + i * NS) * EB
                pltpu.sync_copy(perm_hbm.at[pl.ds(off, EB)], idx_v)
                pltpu.sync_copy(hb_hbm.at[pl.ds(off, EB)], rows_v)
                pltpu.sync_copy(rows_v, acc.at[idx_v], add=True)
                return c

            lax.fori_loop(0, nb_w, body, 0)

        plsc.subcore_barrier()

        @pl.when(cid == 0)
        def _():
            pltpu.sync_copy(acc.at[pl.ds(sid * rpt, rpt)],
                            out_hbm.at[pl.ds(sid * rpt, rpt)])

    return k(hb, perm)


_BM = 256


def _mm_call(A, W, A2=None, W2=None, C=None, rs=None, dinv=None):
    m_pad, Ka = A.shape
    N = W.shape[1]
    grid = (m_pad // _BM,)
    ins = [A, W]
    specs = [pl.BlockSpec((_BM, Ka), lambda i: (i, 0)),
             pl.BlockSpec((Ka, N), lambda i: (0, 0))]
    if A2 is not None:
        Kb = A2.shape[1]
        ins += [A2, W2]
        specs += [pl.BlockSpec((_BM, Kb), lambda i: (i, 0)),
                  pl.BlockSpec((Kb, N), lambda i: (0, 0))]
    if C is not None:
        ins.append(C)
        specs.append(pl.BlockSpec((_BM, N), lambda i: (i, 0)))
    if rs is not None:
        ins.append(rs)
        specs.append(pl.BlockSpec((_BM, 1), lambda i: (i, 0)))
    if dinv is not None:
        ins.append(dinv)
        specs.append(pl.BlockSpec((_BM, 1), lambda i: (i, 0)))
    out_shape = [jax.ShapeDtypeStruct((m_pad, N), F32)]
    out_specs = [pl.BlockSpec((_BM, N), lambda i: (i, 0))]
    if dinv is not None:
        out_shape.append(jax.ShapeDtypeStruct((m_pad, N), F32))
        out_specs.append(pl.BlockSpec((_BM, N), lambda i: (i, 0)))

    def body(*refs):
        it = iter(refs)
        a_ref = next(it)
        w_ref = next(it)
        a2_ref = next(it) if A2 is not None else None
        w2_ref = next(it) if A2 is not None else None
        c_ref = next(it) if C is not None else None
        rs_ref = next(it) if rs is not None else None
        dv_ref = next(it) if dinv is not None else None
        hp_ref = next(it)
        g_ref = next(it) if dinv is not None else None
        a = a_ref[...]
        if rs_ref is not None:
            a = a * jnp.tanh(rs_ref[...])
        h = jnp.dot(a, w_ref[...], preferred_element_type=F32)
        if a2_ref is not None:
            h = h + jnp.dot(a2_ref[...], w2_ref[...],
                            preferred_element_type=F32)
        if c_ref is not None:
            h = h + c_ref[...]
        hp_ref[...] = h
        if g_ref is not None:
            g_ref[...] = h * dv_ref[...]

    res = pl.pallas_call(
        body, grid=grid, in_specs=specs, out_specs=out_specs,
        out_shape=out_shape)(*ins)
    return res if dinv is not None else res[0]


def _dinv_call(deg_part):
    _, m_pad = deg_part.shape

    def body(dp_ref, dv_ref):
        deg = dp_ref[0:1, :] + dp_ref[1:2, :] + 2.0
        dv_ref[...] = lax.rsqrt(deg)

    out = pl.pallas_call(
        body,
        out_shape=jax.ShapeDtypeStruct((1, m_pad), F32))(deg_part)
    return out.reshape(m_pad, 1)


def _epi_call(part, hp, dinv, b, n_rows, act, pvec=None):
    m_pad, N = hp.shape
    grid = (m_pad // _BM,)
    ins = [part, hp, dinv, b.reshape(1, N)]
    specs = [pl.BlockSpec((NC, _BM, N), lambda i: (0, i, 0)),
             pl.BlockSpec((_BM, N), lambda i: (i, 0)),
             pl.BlockSpec((_BM, 1), lambda i: (i, 0)),
             pl.BlockSpec((1, N), lambda i: (0, 0))]
    out_shape = [jax.ShapeDtypeStruct((m_pad, N), F32)]
    out_specs = [pl.BlockSpec((_BM, N), lambda i: (i, 0))]
    if pvec is not None:
        ins.append(pvec.reshape(N, 1))
        specs.append(pl.BlockSpec((N, 1), lambda i: (0, 0)))
        out_shape.append(jax.ShapeDtypeStruct((m_pad, 1), F32))
        out_specs.append(pl.BlockSpec((_BM, 1), lambda i: (i, 0)))

    def body(*refs):
        if pvec is not None:
            part_ref, hp_ref, dv_ref, b_ref, p_ref, out_ref, sc_ref = refs
        else:
            part_ref, hp_ref, dv_ref, b_ref, out_ref = refs
        i = pl.program_id(0)
        s = part_ref[0, :, :] + part_ref[1, :, :]
        d = dv_ref[...]
        v = d * s + (2.0 * d * d) * hp_ref[...] + b_ref[...]
        if act == "relu":
            v = jnp.maximum(v, 0.0)
        elif act == "sigmoid":
            v = jax.nn.sigmoid(v)
        rid = lax.broadcasted_iota(I32, (_BM, 1), 0) + i * _BM
        v = jnp.where(rid < n_rows, v, 0.0)
        out_ref[...] = v
        if pvec is not None:
            pv = p_ref[...]
            pn = lax.rsqrt(jnp.sum(pv * pv))
            sc_ref[...] = jnp.dot(v, pv, preferred_element_type=F32) * pn

    res = pl.pallas_call(
        body, grid=grid, in_specs=specs, out_specs=out_specs,
        out_shape=out_shape)(*ins)
    return res if pvec is not None else res[0]


NNODE = [10000, 5000, 2500, 1250]
NPAD = [10240, 5120, 2560, 1280]


def kernel(x, edge_index, y, W0, b0, W1, b1, W2, b2, W3, b3, p0, p1, p2,
           U0, c0, U1, c1, U2, c2):
    src0 = edge_index[:, 0]
    dst0 = edge_index[:, 1]
    x_pad = jnp.pad(x, ((0, NPAD[0] - NNODE[0]), (0, 0)))

    padw = CAPP - CAP
    junk0 = NNODE[0] + (jnp.arange(padw, dtype=I32) % (NPAD[0] - NNODE[0]))
    junk0 = jnp.broadcast_to(junk0, (NW, padw))

    def pack0(a):
        return jnp.concatenate([a.reshape(NW, CAP), junk0], axis=1).reshape(-1)

    msrc = [pack0(src0), None, None, None]
    mdst = [pack0(dst0), None, None, None]
    cnts = [jnp.full((NW * L,), (CAP + EB - 1) // EB, I32), None, None, None]

    degp0 = _deg_call(mdst[0], cnts[0], NPAD[0])
    dinv0 = _dinv_call(degp0)
    dinvs = [dinv0, None, None, None]
    perms = [None, None, None]
    mems = []

    h = x_pad
    rs = None
    Wd = [(W0, b0), (W1, b1), (W2, b2), (W3, b3)]
    pv = [p0, p1, p2]

    for lvl in range(3):
        Wl, bl = Wd[lvl]
        hp, g = _mm_call(h, Wl, rs=rs, dinv=dinvs[lvl])
        part = _msg_call(g, msrc[lvl], mdst[lvl], cnts[lvl], NPAD[lvl], 128)
        hout, score = _epi_call(part, hp, dinvs[lvl], bl, NNODE[lvl],
                                "relu", pvec=pv[lvl])
        mems.append(hout)
        kk, k_pad = NNODE[lvl + 1], NPAD[lvl + 1]
        vals, perm = lax.top_k(score[:NNODE[lvl], 0], kk)
        perm_p = jnp.pad(perm, (0, k_pad - kk))
        vals_p = jnp.pad(vals, (0, k_pad - kk)).reshape(k_pad, 1)
        perms[lvl] = perm_p
        inv = _inv_call(perm_p, NPAD[lvl], kk, k_pad)
        ms, md, cn = _pool_edges_call(msrc[lvl], mdst[lvl], cnts[lvl], inv,
                                      NPAD[lvl], kk, k_pad)
        msrc[lvl + 1], mdst[lvl + 1], cnts[lvl + 1] = ms, md, cn
        degp = _deg_call(md, cn, k_pad)
        dinvs[lvl + 1] = _dinv_call(degp)
        h = _gather_call(hout, perm_p, k_pad, 128)
        rs = vals_p

    hp, g = _mm_call(h, W3, rs=rs, dinv=dinvs[3])
    part = _msg_call(g, msrc[3], mdst[3], cnts[3], NPAD[3], 128)
    h = _epi_call(part, hp, dinvs[3], b3, NNODE[3], None)

    for lvl, (Uu, cu) in ((2, (U0, c0)), (1, (U1, c1))):
        hb = _mm_call(h, Uu[128:])
        hs = _scatter_call(hb, perms[lvl], NPAD[lvl + 1], NPAD[lvl], 128)
        hp, g = _mm_call(mems[lvl], Uu[:128], C=hs, dinv=dinvs[lvl])
        part = _msg_call(g, msrc[lvl], mdst[lvl], cnts[lvl], NPAD[lvl], 128)
        h = _epi_call(part, hp, dinvs[lvl], cu, NNODE[lvl], "relu")

    U2a = jnp.pad(U2[:128], ((0, 0), (0, 127)))
    U2b = jnp.pad(U2[128:256], ((0, 0), (0, 127)))
    U2c = jnp.pad(U2[256:], ((0, 0), (0, 127)))
    c2p = jnp.pad(c2, (0, 127))
    hb16 = _mm_call(h, U2c)
    hs16 = _scatter_call(hb16, perms[0], NPAD[1], NPAD[0], 128)
    hp16, g16 = _mm_call(mems[0], U2a, A2=x_pad, W2=U2b, C=hs16, dinv=dinv0)
    part16 = _msg_call(g16, msrc[0], mdst[0], cnts[0], NPAD[0], 128)
    out16 = _epi_call(part16, hp16, dinv0, c2p, NNODE[0], "sigmoid")
    return out16[:NNODE[0], 0]


# speedup vs baseline: 68.4031x; 1.0083x over previous
import functools

import jax
import jax.numpy as jnp
from jax import lax
from jax.experimental import pallas as pl
from jax.experimental.pallas import tpu as pltpu
from jax.experimental.pallas import tpu_sc as plsc

F32 = jnp.float32
I32 = jnp.int32
NC, NS, L = 2, 16, 16
NW = NC * NS
EB = 128
E = 320000
CAP = E // NW
CAPP = 12288

_MESH = dict(core_axis_name="c", subcore_axis_name="s", num_cores=NC,
             num_subcores=NS)


def _mesh():
    return plsc.VectorSubcoreMesh(**_MESH)


def _params():
    return pltpu.CompilerParams(needs_layout_passes=False)


def _fill(ref, rows, val, dtype):
    v = jnp.full((L,), val, dtype)

    def body(i, c):
        ref[pl.ds(i * L, L)] = v
        return c

    lax.fori_loop(0, rows // L, body, 0)


def _fill2(ref, rows, D):
    z = jnp.zeros((L,), F32)

    def body(i, c):
        for j in range(D // L):
            ref[i, pl.ds(j * L, L)] = z
        return c

    lax.fori_loop(0, rows, body, 0)


def _deg_call(mdstp, counts, n_pad):
    rpt = n_pad // NS

    @functools.partial(
        pl.kernel,
        out_type=jax.ShapeDtypeStruct((NC * n_pad,), F32),
        mesh=_mesh(),
        compiler_params=_params(),
        scratch_types=[pltpu.VMEM((24, EB), I32), pltpu.VMEM((EB,), F32),
                       pltpu.VMEM((NW * L,), I32), pltpu.VMEM((rpt,), F32),
                       pltpu.SemaphoreType.DMA,
                       pltpu.VMEM_SHARED((n_pad,), F32)],
    )
    def k(dst_hbm, cnt_hbm, out_hbm, idx_v, ones_v, cnt_v, zbuf, sem, acc):
        cid = lax.axis_index("c")
        sid = lax.axis_index("s")
        wid = sid * NC + cid
        _fill(zbuf, rpt, 0.0, F32)
        pltpu.sync_copy(zbuf, acc.at[pl.ds(sid * rpt, rpt)])
        pltpu.sync_copy(cnt_hbm, cnt_v)
        for j in range(EB // L):
            ones_v[pl.ds(j * L, L)] = jnp.full((L,), 1.0, F32)
        plsc.subcore_barrier()
        nb_w = cnt_v[pl.ds(wid * L, L)][0]
        ngr = (nb_w + 24 - 1) // 24

        def group(gi, c):
            base = wid * (CAPP // EB) + gi * 24
            pltpu.sync_copy(dst_hbm.at[pl.ds(base, 24)], idx_v)
            nblk = jnp.minimum(24, nb_w - gi * 24)

            def blk(b, c2):
                pltpu.async_copy(ones_v, acc.at[idx_v.at[b]], sem, add=True)
                return c2

            lax.fori_loop(0, nblk, blk, 0)

            def drain(b, c2):
                pltpu.make_async_copy(ones_v, acc.at[idx_v.at[0]],
                                      sem).wait()
                return c2

            lax.fori_loop(0, nblk, drain, 0)
            return c

        lax.fori_loop(0, ngr, group, 0)
        plsc.subcore_barrier()
        pltpu.sync_copy(acc.at[pl.ds(sid * rpt, rpt)], zbuf)
        pltpu.sync_copy(zbuf, out_hbm.at[pl.ds(cid * n_pad + sid * rpt, rpt)])

    return k(mdstp.reshape(NW * (CAPP // EB), EB), counts).reshape(NC, n_pad)


def _msg_call(g, msrcp, mdstp, counts, n_pad, D):
    rpt = n_pad // NS
    QB = 32
    RB = CAPP // EB

    @functools.partial(
        pl.kernel,
        out_type=jax.ShapeDtypeStruct((NC * n_pad, D), F32),
        mesh=_mesh(),
        compiler_params=_params(),
        scratch_types=[pltpu.VMEM((QB, EB), I32), pltpu.VMEM((QB, EB), I32),
                       pltpu.VMEM((2 * EB, D), F32), pltpu.SemaphoreType.DMA,
                       pltpu.SemaphoreType.DMA,
                       pltpu.VMEM((NW * L,), I32),
                       pltpu.VMEM((40, D), F32),
                       pltpu.VMEM_SHARED((n_pad, D), F32)],
    )
    def k(g_hbm, s_hbm, d_hbm, cnt_hbm, out_hbm,
          src_v, dst_v, rows_v, sem, sem_s, cnt_v, zbuf, acc):
        cid = lax.axis_index("c")
        sid = lax.axis_index("s")
        wid = sid * NC + cid
        _fill2(zbuf, 40, D)

        def zbody(c, carry):
            pltpu.sync_copy(zbuf, acc.at[pl.ds(sid * rpt + c * 40, 40)])
            return carry

        lax.fori_loop(0, rpt // 40, zbody, 0)
        pltpu.sync_copy(cnt_hbm, cnt_v)
        plsc.subcore_barrier()
        nb_w = cnt_v[pl.ds(wid * L, L)][0]
        ngr = (nb_w + QB - 1) // QB

        def group(gi, c):
            base = wid * RB + gi * QB
            pltpu.sync_copy(s_hbm.at[pl.ds(base, QB)], src_v)
            pltpu.sync_copy(d_hbm.at[pl.ds(base, QB)], dst_v)
            nblk = jnp.minimum(QB, nb_w - gi * QB)
            pltpu.async_copy(g_hbm.at[src_v.at[0]],
                             rows_v.at[pl.ds(0, EB)], sem)

            def blk(b, c2):
                slot = lax.rem(b, 2) * EB
                nxt = jnp.minimum(b + 1, nblk - 1)
                pltpu.make_async_copy(g_hbm.at[src_v.at[0]],
                                      rows_v.at[pl.ds(0, EB)], sem).wait()

                @pl.when(b >= 1)
                def _():
                    pltpu.make_async_copy(
                        rows_v.at[pl.ds(0, EB)], acc.at[dst_v.at[0]],
                        sem_s).wait()

                pltpu.async_copy(g_hbm.at[src_v.at[nxt]],
                                 rows_v.at[pl.ds(EB - slot, EB)], sem)
                pltpu.async_copy(rows_v.at[pl.ds(slot, EB)],
                                 acc.at[dst_v.at[b]], sem_s, add=True)
                return c2

            lax.fori_loop(0, nblk, blk, 0)
            pltpu.make_async_copy(g_hbm.at[src_v.at[0]],
                                  rows_v.at[pl.ds(0, EB)], sem).wait()
            pltpu.make_async_copy(rows_v.at[pl.ds(0, EB)],
                                  acc.at[dst_v.at[0]], sem_s).wait()
            return c

        lax.fori_loop(0, ngr, group, 0)
        plsc.subcore_barrier()
        pltpu.sync_copy(acc.at[pl.ds(sid * rpt, rpt)],
                        out_hbm.at[pl.ds(cid * n_pad + sid * rpt, rpt)])

    ms2 = msrcp.reshape(NW * RB, EB)
    md2 = mdstp.reshape(NW * RB, EB)
    return k(g, ms2, md2, counts).reshape(NC, n_pad, D)


def _pool_edges_call(msrcp, mdstp, counts, inv, n_pad, kk, k_pad):
    spread = k_pad - kk

    @functools.partial(
        pl.kernel,
        out_type=(jax.ShapeDtypeStruct((NW * CAPP,), I32),
                  jax.ShapeDtypeStruct((NW * CAPP,), I32),
                  jax.ShapeDtypeStruct((NW * L,), I32)),
        mesh=_mesh(),
        compiler_params=_params(),
        scratch_types=[pltpu.VMEM((n_pad,), I32),
                       pltpu.VMEM((EB,), I32), pltpu.VMEM((EB,), I32),
                       pltpu.VMEM((CAPP,), I32), pltpu.VMEM((CAPP,), I32),
                       pltpu.VMEM((NW * L,), I32), pltpu.VMEM((L,), I32)],
    )
    def k(s_hbm, d_hbm, cnt_hbm, inv_hbm, ms_hbm, md_hbm, cout_hbm,
          inv_v, src_v, dst_v, ms_v, md_v, cnt_v, cb_v):
        cid = lax.axis_index("c")
        sid = lax.axis_index("s")
        wid = sid * NC + cid
        pltpu.sync_copy(inv_hbm, inv_v)
        pltpu.sync_copy(cnt_hbm, cnt_v)
        nb_in = cnt_v[pl.ds(wid * L, L)][0]
        kk_v = jnp.full((L,), kk, I32)
        iota = lax.iota(I32, L)
        junk = kk_v + lax.rem(iota, jnp.full((L,), spread, I32))

        def body(i, off):
            blk = wid * CAPP + i * EB
            pltpu.sync_copy(s_hbm.at[pl.ds(blk, EB)], src_v)
            pltpu.sync_copy(d_hbm.at[pl.ds(blk, EB)], dst_v)
            for j in range(EB // L):
                s16 = src_v[pl.ds(j * L, L)]
                d16 = dst_v[pl.ds(j * L, L)]
                is16 = plsc.load_gather(inv_v, [s16])
                id16 = plsc.load_gather(inv_v, [d16])
                live = (is16 < kk_v) & (id16 < kk_v)
                plsc.store_compressed(ms_v.at[pl.ds(off, L)], is16, mask=live)
                plsc.store_compressed(md_v.at[pl.ds(off, L)], id16, mask=live)
                off = off + jnp.max(plsc.all_reduce_population_count(live))
            return off

        off = lax.fori_loop(0, nb_in, body, 0)
        for t in range(EB // L):
            ms_v[pl.ds(off + t * L, L)] = junk
            md_v[pl.ds(off + t * L, L)] = junk
        nb_out = (off + EB - 1) // EB
        cb_v[pl.ds(0, L)] = jnp.full((L,), 0, I32) + nb_out
        pltpu.sync_copy(ms_v, ms_hbm.at[pl.ds(wid * CAPP, CAPP)])
        pltpu.sync_copy(md_v, md_hbm.at[pl.ds(wid * CAPP, CAPP)])
        pltpu.sync_copy(cb_v, cout_hbm.at[pl.ds(wid * L, L)])

    return k(msrcp, mdstp, counts, inv)


def _inv_call(perm, n_pad, kk, k_pad):
    @functools.partial(
        pl.kernel,
        out_type=jax.ShapeDtypeStruct((n_pad,), I32),
        mesh=_mesh(),
        compiler_params=_params(),
        scratch_types=[pltpu.VMEM((k_pad,), I32), pltpu.VMEM((n_pad,), I32)],
    )
    def k(perm_hbm, out_hbm, perm_v, inv_v):
        cid = lax.axis_index("c")
        sid = lax.axis_index("s")
        wid = sid * NC + cid

        @pl.when(wid == 0)
        def _():
            pltpu.sync_copy(perm_hbm, perm_v)
            fill = jnp.full((L,), kk, I32)

            def fbody(i, c):
                inv_v[pl.ds(i * L, L)] = fill
                return c

            lax.fori_loop(0, n_pad // L, fbody, 0)
            iota = lax.iota(I32, L)

            def sbody(j, c):
                base = j * L
                p16 = perm_v[pl.ds(base, L)]
                vals = iota + base
                mask = vals < kk
                plsc.store_scatter(inv_v, [p16], vals, mask=mask)
                return c

            lax.fori_loop(0, k_pad // L, sbody, 0)
            pltpu.sync_copy(inv_v, out_hbm)

    return k(perm)


def _gather_call(h, perm, k_pad, D):
    nb = k_pad // EB

    @functools.partial(
        pl.kernel,
        out_type=jax.ShapeDtypeStruct((k_pad, D), F32),
        mesh=_mesh(),
        compiler_params=_params(),
        scratch_types=[pltpu.VMEM((EB,), I32), pltpu.VMEM((EB, D), F32),
                       pltpu.SemaphoreType.DMA],
    )
    def k(h_hbm, perm_hbm, out_hbm, idx_v, rows_v, sem):
        cid = lax.axis_index("c")
        sid = lax.axis_index("s")
        wid = sid * NC + cid
        nb_w = (nb - wid + NW - 1) // NW

        def body(i, c):
            off = (wid + i * NW) * EB
            pltpu.sync_copy(perm_hbm.at[pl.ds(off, EB)], idx_v)
            pltpu.async_copy(h_hbm.at[idx_v], rows_v, sem).wait()
            pltpu.sync_copy(rows_v, out_hbm.at[pl.ds(off, EB)])
            return c

        lax.fori_loop(0, nb_w, body, 0)

    return k(h, perm)


def _scatter_call(hb, perm, k_pad, n_pad, D):
    nb = k_pad // EB
    rpt = n_pad // NS

    @functools.partial(
        pl.kernel,
        out_type=jax.ShapeDtypeStruct((n_pad, D), F32),
        mesh=_mesh(),
        compiler_params=_params(),
        scratch_types=[pltpu.VMEM((EB,), I32), pltpu.VMEM((EB, D), F32),
                       pltpu.VMEM((40, D), F32),
                       pltpu.VMEM_SHARED((n_pad, D), F32)],
    )
    def k(hb_hbm, perm_hbm, out_hbm, idx_v, rows_v, zbuf, acc):
        cid = lax.axis_index("c")
        sid = lax.axis_index("s")

        @pl.when(cid == 0)
        def _():
            _fill2(zbuf, 40, D)

            def zbody(c, carry):
                pltpu.sync_copy(zbuf, acc.at[pl.ds(sid * rpt + c * 40, 40)])
                return carry

            lax.fori_loop(0, rpt // 40, zbody, 0)

        plsc.subcore_barrier()

        @pl.when(cid == 0)
        def _():
            nb_w = (nb - sid + NS - 1) // NS

            def body(i, c):
                off = (sid + i * NS) * EB
                pltpu.sync_copy(perm_hbm.at[pl.ds(off, EB)], idx_v)
                pltpu.sync_copy(hb_hbm.at[pl.ds(off, EB)], rows_v)
                pltpu.sync_copy(rows_v, acc.at[idx_v], add=True)
                return c

            lax.fori_loop(0, nb_w, body, 0)

        plsc.subcore_barrier()

        @pl.when(cid == 0)
        def _():
            pltpu.sync_copy(acc.at[pl.ds(sid * rpt, rpt)],
                            out_hbm.at[pl.ds(sid * rpt, rpt)])

    return k(hb, perm)


_BM = 256


def _mm_call(A, W, A2=None, W2=None, C=None, rs=None, dinv=None):
    m_pad, Ka = A.shape
    N = W.shape[1]
    grid = (m_pad // _BM,)
    ins = [A, W]
    specs = [pl.BlockSpec((_BM, Ka), lambda i: (i, 0)),
             pl.BlockSpec((Ka, N), lambda i: (0, 0))]
    if A2 is not None:
        Kb = A2.shape[1]
        ins += [A2, W2]
        specs += [pl.BlockSpec((_BM, Kb), lambda i: (i, 0)),
                  pl.BlockSpec((Kb, N), lambda i: (0, 0))]
    if C is not None:
        ins.append(C)
        specs.append(pl.BlockSpec((_BM, N), lambda i: (i, 0)))
    if rs is not None:
        ins.append(rs)
        specs.append(pl.BlockSpec((_BM, 1), lambda i: (i, 0)))
    if dinv is not None:
        ins.append(dinv)
        specs.append(pl.BlockSpec((_BM, 1), lambda i: (i, 0)))
    out_shape = [jax.ShapeDtypeStruct((m_pad, N), F32)]
    out_specs = [pl.BlockSpec((_BM, N), lambda i: (i, 0))]
    if dinv is not None:
        out_shape.append(jax.ShapeDtypeStruct((m_pad, N), F32))
        out_specs.append(pl.BlockSpec((_BM, N), lambda i: (i, 0)))

    def body(*refs):
        it = iter(refs)
        a_ref = next(it)
        w_ref = next(it)
        a2_ref = next(it) if A2 is not None else None
        w2_ref = next(it) if A2 is not None else None
        c_ref = next(it) if C is not None else None
        rs_ref = next(it) if rs is not None else None
        dv_ref = next(it) if dinv is not None else None
        hp_ref = next(it)
        g_ref = next(it) if dinv is not None else None
        a = a_ref[...]
        if rs_ref is not None:
            a = a * jnp.tanh(rs_ref[...])
        h = jnp.dot(a, w_ref[...], preferred_element_type=F32)
        if a2_ref is not None:
            h = h + jnp.dot(a2_ref[...], w2_ref[...],
                            preferred_element_type=F32)
        if c_ref is not None:
            h = h + c_ref[...]
        hp_ref[...] = h
        if g_ref is not None:
            g_ref[...] = h * dv_ref[...]

    res = pl.pallas_call(
        body, grid=grid, in_specs=specs, out_specs=out_specs,
        out_shape=out_shape)(*ins)
    return res if dinv is not None else res[0]


def _dinv_call(deg_part):
    _, m_pad = deg_part.shape

    def body(dp_ref, dv_ref):
        deg = dp_ref[0:1, :] + dp_ref[1:2, :] + 2.0
        dv_ref[...] = lax.rsqrt(deg)

    out = pl.pallas_call(
        body,
        out_shape=jax.ShapeDtypeStruct((1, m_pad), F32))(deg_part)
    return out.reshape(m_pad, 1)


def _epi_call(part, hp, dinv, b, n_rows, act, pvec=None):
    m_pad, N = hp.shape
    grid = (m_pad // _BM,)
    ins = [part, hp, dinv, b.reshape(1, N)]
    specs = [pl.BlockSpec((NC, _BM, N), lambda i: (0, i, 0)),
             pl.BlockSpec((_BM, N), lambda i: (i, 0)),
             pl.BlockSpec((_BM, 1), lambda i: (i, 0)),
             pl.BlockSpec((1, N), lambda i: (0, 0))]
    out_shape = [jax.ShapeDtypeStruct((m_pad, N), F32)]
    out_specs = [pl.BlockSpec((_BM, N), lambda i: (i, 0))]
    if pvec is not None:
        ins.append(pvec.reshape(N, 1))
        specs.append(pl.BlockSpec((N, 1), lambda i: (0, 0)))
        out_shape.append(jax.ShapeDtypeStruct((m_pad, 1), F32))
        out_specs.append(pl.BlockSpec((_BM, 1), lambda i: (i, 0)))

    def body(*refs):
        if pvec is not None:
            part_ref, hp_ref, dv_ref, b_ref, p_ref, out_ref, sc_ref = refs
        else:
            part_ref, hp_ref, dv_ref, b_ref, out_ref = refs
        i = pl.program_id(0)
        s = part_ref[0, :, :] + part_ref[1, :, :]
        d = dv_ref[...]
        v = d * s + (2.0 * d * d) * hp_ref[...] + b_ref[...]
        if act == "relu":
            v = jnp.maximum(v, 0.0)
        elif act == "sigmoid":
            v = jax.nn.sigmoid(v)
        rid = lax.broadcasted_iota(I32, (_BM, 1), 0) + i * _BM
        v = jnp.where(rid < n_rows, v, 0.0)
        out_ref[...] = v
        if pvec is not None:
            pv = p_ref[...]
            pn = lax.rsqrt(jnp.sum(pv * pv))
            sc_ref[...] = jnp.dot(v, pv, preferred_element_type=F32) * pn

    res = pl.pallas_call(
        body, grid=grid, in_specs=specs, out_specs=out_specs,
        out_shape=out_shape)(*ins)
    return res if pvec is not None else res[0]


NNODE = [10000, 5000, 2500, 1250]
NPAD = [10240, 5120, 2560, 1280]


def kernel(x, edge_index, y, W0, b0, W1, b1, W2, b2, W3, b3, p0, p1, p2,
           U0, c0, U1, c1, U2, c2):
    src0 = edge_index[:, 0]
    dst0 = edge_index[:, 1]
    x_pad = jnp.pad(x, ((0, NPAD[0] - NNODE[0]), (0, 0)))

    padw = CAPP - CAP
    junk0 = NNODE[0] + (jnp.arange(padw, dtype=I32) % (NPAD[0] - NNODE[0]))
    junk0 = jnp.broadcast_to(junk0, (NW, padw))

    def pack0(a):
        return jnp.concatenate([a.reshape(NW, CAP), junk0], axis=1).reshape(-1)

    msrc = [pack0(src0), None, None, None]
    mdst = [pack0(dst0), None, None, None]
    cnts = [jnp.full((NW * L,), (CAP + EB - 1) // EB, I32), None, None, None]

    degp0 = _deg_call(mdst[0], cnts[0], NPAD[0])
    dinv0 = _dinv_call(degp0)
    dinvs = [dinv0, None, None, None]
    perms = [None, None, None]
    mems = []

    h = x_pad
    rs = None
    Wd = [(W0, b0), (W1, b1), (W2, b2), (W3, b3)]
    pv = [p0, p1, p2]

    for lvl in range(3):
        Wl, bl = Wd[lvl]
        hp, g = _mm_call(h, Wl, rs=rs, dinv=dinvs[lvl])
        part = _msg_call(g, msrc[lvl], mdst[lvl], cnts[lvl], NPAD[lvl], 128)
        hout, score = _epi_call(part, hp, dinvs[lvl], bl, NNODE[lvl],
                                "relu", pvec=pv[lvl])
        mems.append(hout)
        kk, k_pad = NNODE[lvl + 1], NPAD[lvl + 1]
        vals, perm = lax.top_k(score[:NNODE[lvl], 0], kk)
        perm_p = jnp.pad(perm, (0, k_pad - kk))
        vals_p = jnp.pad(vals, (0, k_pad - kk)).reshape(k_pad, 1)
        perms[lvl] = perm_p
        inv = _inv_call(perm_p, NPAD[lvl], kk, k_pad)
        ms, md, cn = _pool_edges_call(msrc[lvl], mdst[lvl], cnts[lvl], inv,
                                      NPAD[lvl], kk, k_pad)
        msrc[lvl + 1], mdst[lvl + 1], cnts[lvl + 1] = ms, md, cn
        degp = _deg_call(md, cn, k_pad)
        dinvs[lvl + 1] = _dinv_call(degp)
        h = _gather_call(hout, perm_p, k_pad, 128)
        rs = vals_p

    hp, g = _mm_call(h, W3, rs=rs, dinv=dinvs[3])
    part = _msg_call(g, msrc[3], mdst[3], cnts[3], NPAD[3], 128)
    h = _epi_call(part, hp, dinvs[3], b3, NNODE[3], None)

    for lvl, (Uu, cu) in ((2, (U0, c0)), (1, (U1, c1))):
        hb = _mm_call(h, Uu[128:])
        hs = _scatter_call(hb, perms[lvl], NPAD[lvl + 1], NPAD[lvl], 128)
        hp, g = _mm_call(mems[lvl], Uu[:128], C=hs, dinv=dinvs[lvl])
        part = _msg_call(g, msrc[lvl], mdst[lvl], cnts[lvl], NPAD[lvl], 128)
        h = _epi_call(part, hp, dinvs[lvl], cu, NNODE[lvl], "relu")

    U2a = jnp.pad(U2[:128], ((0, 0), (0, 127)))
    U2b = jnp.pad(U2[128:256], ((0, 0), (0, 127)))
    U2c = jnp.pad(U2[256:], ((0, 0), (0, 127)))
    c2p = jnp.pad(c2, (0, 127))
    hb16 = _mm_call(h, U2c)
    hs16 = _scatter_call(hb16, perms[0], NPAD[1], NPAD[0], 128)
    hp16, g16 = _mm_call(mems[0], U2a, A2=x_pad, W2=U2b, C=hs16, dinv=dinv0)
    part16 = _msg_call(g16, msrc[0], mdst[0], cnts[0], NPAD[0], 128)
    out16 = _epi_call(part16, hp16, dinv0, c2p, NNODE[0], "sigmoid")
    return out16[:NNODE[0], 0]
